# Initial kernel scaffold; baseline (speedup 1.0000x reference)
import jax
import jax.numpy as jnp
from jax.experimental import pallas as pl


def kernel(query, reference_points, value, spatial_shapes, level_start_index, W_off, b_off, W_attn, b_attn, W_val, b_val, W_out, b_out):
    raise NotImplementedError("write your pallas kernel here")


# speedup vs baseline: 67.6749x; 67.6749x over previous
import functools

import jax
import jax.numpy as jnp
import numpy as np
from jax import lax
from jax.experimental import pallas as pl
from jax.experimental.pallas import tpu as pltpu
from jax.experimental.pallas import tpu_sc as plsc

_SS = np.array([[64, 64], [32, 32], [16, 16], [8, 8]], dtype=np.int64)
_AREAS = _SS[:, 0] * _SS[:, 1]
_LSI = np.concatenate([np.zeros(1, dtype=np.int64), np.cumsum(_AREAS)[:-1]])
LV = int(_AREAS.sum())
LQ = LV
B = 4
C = 256
NH, NL, NP = 8, 4, 4
D = C // NH

QT = 680
NQT = LQ // QT

NC, NS = 2, 16
NW = NC * NS
GROUPS = B * LQ * NH
GPW = GROUPS // NW
G = 16
STEPS = GPW // G
RPS = G * NL * NP * 4
IDX_ROWS = RPS // 128

def _lane_consts():
    kio = lax.broadcasted_iota(jnp.int32, (1, 16), 1)
    lidx = kio // NP
    wrow_i = lax.shift_right_logical(jnp.full((1, 16), 64, jnp.int32), lidx)
    wrow = wrow_i.astype(jnp.float32)
    hrow = wrow
    base8 = jnp.where(
        lidx == 0, 0,
        jnp.where(lidx == 1, int(_LSI[1]) * NH,
                  jnp.where(lidx == 2, int(_LSI[2]) * NH, int(_LSI[3]) * NH)))
    return wrow, hrow, wrow_i, base8


def _valproj_body(v_ref, w_ref, b_ref, o_ref):
    o_ref[...] = jnp.dot(v_ref[...], w_ref[...],
                         preferred_element_type=jnp.float32) + b_ref[...]


def _outproj_body(x_ref, w_ref, b_ref, o_ref):
    o_ref[...] = jnp.dot(x_ref[...], w_ref[...],
                         preferred_element_type=jnp.float32) + b_ref[...]


def _sample_body(q_ref, rp_ref, woff_ref, boff_ref, wattn_ref, battn_ref,
                 idx_ref, w_ref):
    b = pl.program_id(0)
    q = q_ref[0]
    off = jnp.dot(q, woff_ref[...],
                  preferred_element_type=jnp.float32) + boff_ref[...]
    attn = jnp.dot(q, wattn_ref[...],
                   preferred_element_type=jnp.float32) + battn_ref[...]
    rp = rp_ref[0]
    rpx = rp[:, :16]
    rpy = rp[:, 16:]
    _WROW, _HROW, _WROWI, _BASE8 = _lane_consts()
    idx_parts = []
    w_parts = []
    for h in range(NH):
        ah = attn[:, h * 16:(h + 1) * 16]
        m = jnp.max(ah, axis=1, keepdims=True)
        e = jnp.exp(ah - m)
        aw = e / jnp.sum(e, axis=1, keepdims=True)
        oh = off[:, h * 32:(h + 1) * 32]
        x = rpx * _WROW + oh[:, :16] - 0.5
        y = rpy * _HROW + oh[:, 16:] - 0.5
        x0 = jnp.floor(x)
        y0 = jnp.floor(y)
        fx = x - x0
        fy = y - y0
        vx0 = (x0 >= 0.0) & (x0 <= _WROW - 1.0)
        vx1 = (x0 + 1.0 >= 0.0) & (x0 + 1.0 <= _WROW - 1.0)
        vy0 = (y0 >= 0.0) & (y0 <= _HROW - 1.0)
        vy1 = (y0 + 1.0 >= 0.0) & (y0 + 1.0 <= _HROW - 1.0)
        xc0 = jnp.clip(x0, 0.0, _WROW - 1.0).astype(jnp.int32)
        xc1 = jnp.clip(x0 + 1.0, 0.0, _WROW - 1.0).astype(jnp.int32)
        yc0 = jnp.clip(y0, 0.0, _HROW - 1.0).astype(jnp.int32)
        yc1 = jnp.clip(y0 + 1.0, 0.0, _HROW - 1.0).astype(jnp.int32)
        base = _BASE8 + (b * LQ * NH + h)
        i00 = base + (yc0 * _WROWI + xc0) * NH
        i10 = base + (yc0 * _WROWI + xc1) * NH
        i01 = base + (yc1 * _WROWI + xc0) * NH
        i11 = base + (yc1 * _WROWI + xc1) * NH
        w00 = (1.0 - fx) * (1.0 - fy) * aw * (vx0 & vy0).astype(jnp.float32)
        w10 = fx * (1.0 - fy) * aw * (vx1 & vy0).astype(jnp.float32)
        w01 = (1.0 - fx) * fy * aw * (vx0 & vy1).astype(jnp.float32)
        w11 = fx * fy * aw * (vx1 & vy1).astype(jnp.float32)
        idx_parts += [i00, i10, i01, i11]
        w_parts += [w00, w10, w01, w11]
    idx_ref[0] = jnp.concatenate(idx_parts, axis=1)
    w_ref[0] = jnp.concatenate(w_parts, axis=1)


def _sc_body(table, idx_hbm, w_hbm, out_hbm, idx_v, w_v, rows_v, out_v, sem):
    wid = lax.axis_index("s") * NC + lax.axis_index("c")
    base_group = wid * GPW

    def step(s, carry):
        g0 = pl.multiple_of(base_group + s * G, G)
        pltpu.sync_copy(idx_hbm.at[pl.ds(pl.multiple_of(g0 // 2, 8), IDX_ROWS)],
                        idx_v)
        pltpu.sync_copy(w_hbm.at[pl.ds(pl.multiple_of(g0 * 64, 128), G * 64)],
                        w_v)
        descs = []
        for j in range(IDX_ROWS):
            descs.append(pltpu.async_copy(
                table.at[idx_v.at[j]], rows_v.at[pl.ds(j * 128, 128)], sem))
        for dsc in descs:
            dsc.wait()

        def group(g, c2):
            acc0 = jnp.zeros((16,), jnp.float32)
            acc1 = jnp.zeros((16,), jnp.float32)
            gbase = jnp.full((16,), g * 64, jnp.int32)
            for i in range(NL * NP * 4):
                ws = plsc.load_gather(w_v, [gbase + i])
                r = g * 64 + i
                acc0 = acc0 + rows_v[r, pl.ds(0, 16)] * ws
                acc1 = acc1 + rows_v[r, pl.ds(16, 16)] * ws
            out_v[g, pl.ds(0, 16)] = acc0
            out_v[g, pl.ds(16, 16)] = acc1
            return c2

        lax.fori_loop(0, G, group, 0)
        pltpu.sync_copy(out_v, out_hbm.at[pl.ds(g0, G)])
        return carry

    lax.fori_loop(0, STEPS, step, 0)


def kernel(query, reference_points, value, spatial_shapes, level_start_index,
           W_off, b_off, W_attn, b_attn, W_val, b_val, W_out, b_out):
    f32 = jnp.float32

    valproj = pl.pallas_call(
        _valproj_body,
        grid=(B * LV // QT,),
        in_specs=[
            pl.BlockSpec((QT, C), lambda i: (i, 0)),
            pl.BlockSpec((C, C), lambda i: (0, 0)),
            pl.BlockSpec((1, C), lambda i: (0, 0)),
        ],
        out_specs=pl.BlockSpec((QT, C), lambda i: (i, 0)),
        out_shape=jax.ShapeDtypeStruct((B * LV, C), f32),
    )(value.reshape(B * LV, C), W_val, b_val.reshape(1, C))
    table = valproj.reshape(B * LV * NH, D)

    woffp = W_off.reshape(C, NH, NL, NP, 2).transpose(0, 1, 4, 2, 3)
    woffp = woffp.reshape(C, C)
    boffp = b_off.reshape(NH, NL, NP, 2).transpose(0, 3, 1, 2).reshape(1, C)
    rp32 = jnp.repeat(reference_points.transpose(0, 1, 3, 2), NP, axis=3)
    rp32 = rp32.reshape(B, LQ, 32)
    idx, w = pl.pallas_call(
        _sample_body,
        grid=(B, NQT),
        in_specs=[
            pl.BlockSpec((1, QT, C), lambda b, i: (b, i, 0)),
            pl.BlockSpec((1, QT, 32), lambda b, i: (b, i, 0)),
            pl.BlockSpec((C, C), lambda b, i: (0, 0)),
            pl.BlockSpec((1, C), lambda b, i: (0, 0)),
            pl.BlockSpec((C, NH * 16), lambda b, i: (0, 0)),
            pl.BlockSpec((1, NH * 16), lambda b, i: (0, 0)),
        ],
        out_specs=[
            pl.BlockSpec((1, QT, 512), lambda b, i: (b, i, 0)),
            pl.BlockSpec((1, QT, 512), lambda b, i: (b, i, 0)),
        ],
        out_shape=[
            jax.ShapeDtypeStruct((B, LQ, 512), jnp.int32),
            jax.ShapeDtypeStruct((B, LQ, 512), f32),
        ],
    )(query, rp32, woffp, boffp, W_attn, b_attn.reshape(1, NH * 16))
    idx2d = idx.reshape(GROUPS * 64 // 128, 128)
    w2d = w.reshape(GROUPS * 64)

    mesh = plsc.VectorSubcoreMesh(core_axis_name="c", subcore_axis_name="s",
                                  num_cores=NC, num_subcores=NS)
    sc = pl.kernel(
        _sc_body,
        out_type=jax.ShapeDtypeStruct((GROUPS, D), f32),
        mesh=mesh,
        compiler_params=pltpu.CompilerParams(needs_layout_passes=False,
                                             use_tc_tiling_on_sc=False),
        scratch_types=[
            pltpu.VMEM((IDX_ROWS, 128), jnp.int32),
            pltpu.VMEM((G * 64,), f32),
            pltpu.VMEM((RPS, D), f32),
            pltpu.VMEM((G, D), f32),
            pltpu.SemaphoreType.DMA,
        ],
    )
    sampled = sc(table, idx2d, w2d)

    out = pl.pallas_call(
        _outproj_body,
        grid=(B * LQ // QT,),
        in_specs=[
            pl.BlockSpec((QT, C), lambda i: (i, 0)),
            pl.BlockSpec((C, C), lambda i: (0, 0)),
            pl.BlockSpec((1, C), lambda i: (0, 0)),
        ],
        out_specs=pl.BlockSpec((QT, C), lambda i: (i, 0)),
        out_shape=jax.ShapeDtypeStruct((B * LQ, C), f32),
    )(sampled.reshape(B * LQ, C), W_out, b_out.reshape(1, C))
    return out.reshape(B, LQ, C)


# speedup vs baseline: 111.0538x; 1.6410x over previous
import functools

import jax
import jax.numpy as jnp
import numpy as np
from jax import lax
from jax.experimental import pallas as pl
from jax.experimental.pallas import tpu as pltpu
from jax.experimental.pallas import tpu_sc as plsc

_SS = np.array([[64, 64], [32, 32], [16, 16], [8, 8]], dtype=np.int64)
_AREAS = _SS[:, 0] * _SS[:, 1]
_LSI = np.concatenate([np.zeros(1, dtype=np.int64), np.cumsum(_AREAS)[:-1]])
LV = int(_AREAS.sum())
LQ = LV
B = 4
C = 256
NH, NL, NP = 8, 4, 4
D = C // NH

QT = 680
NQT = LQ // QT

NC, NS = 2, 16
NW = NC * NS
GROUPS = B * LQ * NH
GPW = GROUPS // NW
G = 16
STEPS = GPW // G
RPS = G * NL * NP * 4
IDX_ROWS = RPS // 128

def _lane_consts():
    kio = lax.broadcasted_iota(jnp.int32, (1, 16), 1)
    lidx = kio // NP
    wrow_i = lax.shift_right_logical(jnp.full((1, 16), 64, jnp.int32), lidx)
    wrow = wrow_i.astype(jnp.float32)
    hrow = wrow
    base8 = jnp.where(
        lidx == 0, 0,
        jnp.where(lidx == 1, int(_LSI[1]) * NH,
                  jnp.where(lidx == 2, int(_LSI[2]) * NH, int(_LSI[3]) * NH)))
    return wrow, hrow, wrow_i, base8


def _valproj_body(v_ref, w_ref, b_ref, o_ref):
    o_ref[...] = jnp.dot(v_ref[...], w_ref[...],
                         preferred_element_type=jnp.float32) + b_ref[...]


def _outproj_body(x_ref, w_ref, b_ref, o_ref):
    o_ref[...] = jnp.dot(x_ref[...], w_ref[...],
                         preferred_element_type=jnp.float32) + b_ref[...]


def _sample_body(q_ref, rp_ref, woff_ref, boff_ref, wattn_ref, battn_ref,
                 idx_ref, w_ref):
    b = pl.program_id(0)
    q = q_ref[0]
    off = jnp.dot(q, woff_ref[...],
                  preferred_element_type=jnp.float32) + boff_ref[...]
    attn = jnp.dot(q, wattn_ref[...],
                   preferred_element_type=jnp.float32) + battn_ref[...]
    rp = rp_ref[0]
    rpx = rp[:, :16]
    rpy = rp[:, 16:]
    _WROW, _HROW, _WROWI, _BASE8 = _lane_consts()
    idx_parts = []
    w_parts = []
    for h in range(NH):
        ah = attn[:, h * 16:(h + 1) * 16]
        m = jnp.max(ah, axis=1, keepdims=True)
        e = jnp.exp(ah - m)
        aw = e / jnp.sum(e, axis=1, keepdims=True)
        oh = off[:, h * 32:(h + 1) * 32]
        x = rpx * _WROW + oh[:, :16] - 0.5
        y = rpy * _HROW + oh[:, 16:] - 0.5
        x0 = jnp.floor(x)
        y0 = jnp.floor(y)
        fx = x - x0
        fy = y - y0
        vx0 = (x0 >= 0.0) & (x0 <= _WROW - 1.0)
        vx1 = (x0 + 1.0 >= 0.0) & (x0 + 1.0 <= _WROW - 1.0)
        vy0 = (y0 >= 0.0) & (y0 <= _HROW - 1.0)
        vy1 = (y0 + 1.0 >= 0.0) & (y0 + 1.0 <= _HROW - 1.0)
        xc0 = jnp.clip(x0, 0.0, _WROW - 1.0).astype(jnp.int32)
        xc1 = jnp.clip(x0 + 1.0, 0.0, _WROW - 1.0).astype(jnp.int32)
        yc0 = jnp.clip(y0, 0.0, _HROW - 1.0).astype(jnp.int32)
        yc1 = jnp.clip(y0 + 1.0, 0.0, _HROW - 1.0).astype(jnp.int32)
        base = _BASE8 + (b * LQ * NH + h)
        i00 = base + (yc0 * _WROWI + xc0) * NH
        i10 = base + (yc0 * _WROWI + xc1) * NH
        i01 = base + (yc1 * _WROWI + xc0) * NH
        i11 = base + (yc1 * _WROWI + xc1) * NH
        w00 = (1.0 - fx) * (1.0 - fy) * aw * (vx0 & vy0).astype(jnp.float32)
        w10 = fx * (1.0 - fy) * aw * (vx1 & vy0).astype(jnp.float32)
        w01 = (1.0 - fx) * fy * aw * (vx0 & vy1).astype(jnp.float32)
        w11 = fx * fy * aw * (vx1 & vy1).astype(jnp.float32)
        idx_parts += [i00, i10, i01, i11]
        w_parts += [w00, w10, w01, w11]
    idx_ref[0] = jnp.concatenate(idx_parts, axis=1)
    w_ref[0] = jnp.concatenate(w_parts, axis=1)


def _sc_body(table, idx_hbm, w_hbm, out_hbm,
             m0, m1, m2, m3, w0, w1, w2, w3, r0, r1, o0, o1,
             smi0, smi1, smi2, smi3, smw0, smw1, smw2, smw3,
             sg0, sg1, so0, so1):
    wid = lax.axis_index("s") * NC + lax.axis_index("c")
    ms = [m0, m1, m2, m3]
    wv = [w0, w1, w2, w3]
    rs = [r0, r1]
    os = [o0, o1]
    smi = [smi0, smi1, smi2, smi3]
    smw = [smw0, smw1, smw2, smw3]
    sg = [sg0, sg1]
    so = [so0, so1]
    ibase = wid * (GPW * 64 // 128)
    wbase = wid * (GPW * 64)
    obase = wid * GPW

    def idx_src(s):
        s = jnp.minimum(s, STEPS - 1)
        return idx_hbm.at[pl.ds(pl.multiple_of(ibase + s * IDX_ROWS, 8),
                                IDX_ROWS)]

    def w_src(s):
        s = jnp.minimum(s, STEPS - 1)
        return w_hbm.at[pl.ds(pl.multiple_of(wbase + s * RPS, 128), RPS)]

    def fire(mbuf, rbuf, sem):
        for j in range(IDX_ROWS):
            pltpu.async_copy(table.at[mbuf.at[j]],
                             rbuf.at[pl.ds(j * 128, 128)], sem)

    def compute(wbuf, rbuf, obuf):
        def group(g, c2):
            acc0 = jnp.zeros((16,), jnp.float32)
            acc1 = jnp.zeros((16,), jnp.float32)
            gbase = jnp.full((16,), g * 64, jnp.int32)
            for i in range(NL * NP * 4):
                s = plsc.load_gather(wbuf, [gbase + i])
                r = g * 64 + i
                acc0 = acc0 + rbuf[r, pl.ds(0, 16)] * s
                acc1 = acc1 + rbuf[r, pl.ds(16, 16)] * s
            obuf[g, pl.ds(0, 16)] = acc0
            obuf[g, pl.ds(16, 16)] = acc1
            return c2

        lax.fori_loop(0, G, group, 0)

    pltpu.async_copy(idx_src(0), ms[0], smi[0])
    pltpu.async_copy(w_src(0), wv[0], smw[0])
    pltpu.async_copy(idx_src(1), ms[1], smi[1])
    pltpu.async_copy(w_src(1), wv[1], smw[1])
    pltpu.make_async_copy(idx_src(0), ms[0], smi[0]).wait()
    fire(ms[0], rs[0], sg[0])

    def iter_t(t, carry):
        s0 = t * 4
        for k in range(4):
            s = s0 + k
            p = k % 2
            ka, kb = (k + 1) % 4, (k + 2) % 4
            pltpu.make_async_copy(idx_src(s + 1), ms[ka], smi[ka]).wait()
            fire(ms[ka], rs[1 - p], sg[1 - p])
            pltpu.async_copy(idx_src(s + 2), ms[kb], smi[kb])
            pltpu.async_copy(w_src(s + 2), wv[kb], smw[kb])
            pltpu.make_async_copy(table.at[pl.ds(0, RPS)], rs[p], sg[p]).wait()
            pltpu.make_async_copy(w_src(s), wv[k], smw[k]).wait()

            @pl.when(s >= 2)
            def _():
                pltpu.make_async_copy(os[p], out_hbm.at[pl.ds(0, G)],
                                      so[p]).wait()

            compute(wv[k], rs[p], os[p])
            pltpu.async_copy(
                os[p],
                out_hbm.at[pl.ds(pl.multiple_of(obase + s * G, 8), G)], so[p])
        return carry

    lax.fori_loop(0, STEPS // 4, iter_t, 0)
    pltpu.make_async_copy(table.at[pl.ds(0, RPS)], rs[0], sg[0]).wait()
    pltpu.make_async_copy(idx_src(0), ms[1], smi[1]).wait()
    pltpu.make_async_copy(w_src(0), wv[0], smw[0]).wait()
    pltpu.make_async_copy(w_src(0), wv[1], smw[1]).wait()
    pltpu.make_async_copy(os[0], out_hbm.at[pl.ds(0, G)], so[0]).wait()
    pltpu.make_async_copy(os[1], out_hbm.at[pl.ds(0, G)], so[1]).wait()


def kernel(query, reference_points, value, spatial_shapes, level_start_index,
           W_off, b_off, W_attn, b_attn, W_val, b_val, W_out, b_out):
    f32 = jnp.float32

    valproj = pl.pallas_call(
        _valproj_body,
        grid=(B * LV // QT,),
        in_specs=[
            pl.BlockSpec((QT, C), lambda i: (i, 0)),
            pl.BlockSpec((C, C), lambda i: (0, 0)),
            pl.BlockSpec((1, C), lambda i: (0, 0)),
        ],
        out_specs=pl.BlockSpec((QT, C), lambda i: (i, 0)),
        out_shape=jax.ShapeDtypeStruct((B * LV, C), f32),
    )(value.reshape(B * LV, C), W_val, b_val.reshape(1, C))
    table = valproj.reshape(B * LV * NH, D)

    woffp = W_off.reshape(C, NH, NL, NP, 2).transpose(0, 1, 4, 2, 3)
    woffp = woffp.reshape(C, C)
    boffp = b_off.reshape(NH, NL, NP, 2).transpose(0, 3, 1, 2).reshape(1, C)
    rp32 = jnp.repeat(reference_points.transpose(0, 1, 3, 2), NP, axis=3)
    rp32 = rp32.reshape(B, LQ, 32)
    idx, w = pl.pallas_call(
        _sample_body,
        grid=(B, NQT),
        in_specs=[
            pl.BlockSpec((1, QT, C), lambda b, i: (b, i, 0)),
            pl.BlockSpec((1, QT, 32), lambda b, i: (b, i, 0)),
            pl.BlockSpec((C, C), lambda b, i: (0, 0)),
            pl.BlockSpec((1, C), lambda b, i: (0, 0)),
            pl.BlockSpec((C, NH * 16), lambda b, i: (0, 0)),
            pl.BlockSpec((1, NH * 16), lambda b, i: (0, 0)),
        ],
        out_specs=[
            pl.BlockSpec((1, QT, 512), lambda b, i: (b, i, 0)),
            pl.BlockSpec((1, QT, 512), lambda b, i: (b, i, 0)),
        ],
        out_shape=[
            jax.ShapeDtypeStruct((B, LQ, 512), jnp.int32),
            jax.ShapeDtypeStruct((B, LQ, 512), f32),
        ],
    )(query, rp32, woffp, boffp, W_attn, b_attn.reshape(1, NH * 16))
    idx2d = idx.reshape(GROUPS * 64 // 128, 128)
    w2d = w.reshape(GROUPS * 64)

    mesh = plsc.VectorSubcoreMesh(core_axis_name="c", subcore_axis_name="s",
                                  num_cores=NC, num_subcores=NS)
    sc = pl.kernel(
        _sc_body,
        out_type=jax.ShapeDtypeStruct((GROUPS, D), f32),
        mesh=mesh,
        compiler_params=pltpu.CompilerParams(needs_layout_passes=False,
                                             use_tc_tiling_on_sc=False),
        scratch_types=(
            [pltpu.VMEM((IDX_ROWS, 128), jnp.int32)] * 4
            + [pltpu.VMEM((RPS,), f32)] * 4
            + [pltpu.VMEM((RPS, D), f32)] * 2
            + [pltpu.VMEM((G, D), f32)] * 2
            + [pltpu.SemaphoreType.DMA] * 12
        ),
    )
    sampled = sc(table, idx2d, w2d)

    out = pl.pallas_call(
        _outproj_body,
        grid=(B * LQ // QT,),
        in_specs=[
            pl.BlockSpec((QT, C), lambda i: (i, 0)),
            pl.BlockSpec((C, C), lambda i: (0, 0)),
            pl.BlockSpec((1, C), lambda i: (0, 0)),
        ],
        out_specs=pl.BlockSpec((QT, C), lambda i: (i, 0)),
        out_shape=jax.ShapeDtypeStruct((B * LQ, C), f32),
    )(sampled.reshape(B * LQ, C), W_out, b_out.reshape(1, C))
    return out.reshape(B, LQ, C)


# speedup vs baseline: 153.9982x; 1.3867x over previous
import functools

import jax
import jax.numpy as jnp
import numpy as np
from jax import lax
from jax.experimental import pallas as pl
from jax.experimental.pallas import tpu as pltpu
from jax.experimental.pallas import tpu_sc as plsc

_SS = np.array([[64, 64], [32, 32], [16, 16], [8, 8]], dtype=np.int64)
_AREAS = _SS[:, 0] * _SS[:, 1]
_LSI = np.concatenate([np.zeros(1, dtype=np.int64), np.cumsum(_AREAS)[:-1]])
LV = int(_AREAS.sum())
LQ = LV
B = 4
C = 256
NH, NL, NP = 8, 4, 4
D = C // NH

QT = 680
NQT = LQ // QT

NC, NS = 2, 16
NW = NC * NS
GROUPS = B * LQ * NH
GPW = GROUPS // NW
G = 16
STEPS = GPW // G
RPS = G * NL * NP * 4
IDX_ROWS = RPS // 128

def _valproj_body(v_ref, w_ref, b_ref, o_ref):
    o_ref[...] = jnp.dot(v_ref[...], w_ref[...],
                         preferred_element_type=jnp.float32) + b_ref[...]


def _outproj_body(x_ref, w_ref, b_ref, o_ref):
    o_ref[...] = jnp.dot(x_ref[...], w_ref[...],
                         preferred_element_type=jnp.float32) + b_ref[...]


def _sample_body(q_ref, rp_ref, woff_ref, boff_ref, wattn_ref, battn_ref,
                 idx_ref, w_ref):
    i32, f32 = jnp.int32, jnp.float32
    b = pl.program_id(0)
    q = q_ref[0]
    off = jnp.dot(q, woff_ref[...],
                  preferred_element_type=f32) + boff_ref[...]
    attn = jnp.dot(q, wattn_ref[...],
                   preferred_element_type=f32) + battn_ref[...]
    m = jnp.max(attn, axis=1, keepdims=True)
    e = jnp.exp(attn - m)
    r0 = lax.broadcasted_iota(i32, (128, 128), 0)
    c0 = lax.broadcasted_iota(i32, (128, 128), 1)
    seg = (r0 // 16 == c0 // 16).astype(f32)
    denom = jnp.dot(e, seg, preferred_element_type=f32)
    aw = e / denom
    rp8 = rp_ref[0]
    kio = lax.broadcasted_iota(i32, (1, 128), 1)
    lidx = (kio % 16) // 4
    h_lane = kio // 16
    wrow_i = lax.shift_right_logical(jnp.full((1, 128), 64, i32), lidx)
    wrow = wrow_i.astype(f32)
    base_c = jnp.where(
        lidx == 0, 0,
        jnp.where(lidx == 1, int(_LSI[1]) * NH,
                  jnp.where(lidx == 2, int(_LSI[2]) * NH,
                            int(_LSI[3]) * NH))) + h_lane
    xb = jnp.zeros((QT, 128), f32)
    yb = jnp.zeros((QT, 128), f32)
    for l in range(NL):
        msk = (lidx == l).astype(f32)
        xb = xb + msk * rp8[:, l:l + 1]
        yb = yb + msk * rp8[:, NL + l:NL + l + 1]
    x = xb + off[:, :128] - 0.5
    y = yb + off[:, 128:] - 0.5
    x0 = jnp.floor(x)
    y0 = jnp.floor(y)
    fx = x - x0
    fy = y - y0
    vx0 = (x0 >= 0.0) & (x0 <= wrow - 1.0)
    vx1 = (x0 + 1.0 >= 0.0) & (x0 + 1.0 <= wrow - 1.0)
    vy0 = (y0 >= 0.0) & (y0 <= wrow - 1.0)
    vy1 = (y0 + 1.0 >= 0.0) & (y0 + 1.0 <= wrow - 1.0)
    xc0 = jnp.clip(x0, 0.0, wrow - 1.0).astype(i32)
    xc1 = jnp.clip(x0 + 1.0, 0.0, wrow - 1.0).astype(i32)
    yc0 = jnp.clip(y0, 0.0, wrow - 1.0).astype(i32)
    yc1 = jnp.clip(y0 + 1.0, 0.0, wrow - 1.0).astype(i32)
    base = base_c + b * (LQ * NH)
    i00 = base + (yc0 * wrow_i + xc0) * NH
    i10 = base + (yc0 * wrow_i + xc1) * NH
    i01 = base + (yc1 * wrow_i + xc0) * NH
    i11 = base + (yc1 * wrow_i + xc1) * NH
    w00 = (1.0 - fx) * (1.0 - fy) * aw * (vx0 & vy0).astype(f32)
    w10 = fx * (1.0 - fy) * aw * (vx1 & vy0).astype(f32)
    w01 = (1.0 - fx) * fy * aw * (vx0 & vy1).astype(f32)
    w11 = fx * fy * aw * (vx1 & vy1).astype(f32)
    icat = jnp.concatenate([t.reshape(QT, 1, 128) for t in
                            (i00, i10, i01, i11)], axis=1)
    wcat = jnp.concatenate([t.reshape(QT, 1, 128) for t in
                            (w00, w10, w01, w11)], axis=1)
    idx_ref[...] = icat.reshape(QT * 4, 128)
    w_ref[...] = wcat.reshape(QT * 4, 128)


def _sc_body(table, idx_hbm, w_hbm, out_hbm,
             m0, m1, m2, m3, w0, w1, w2, w3, r0, r1, o0, o1,
             smi0, smi1, smi2, smi3, smw0, smw1, smw2, smw3,
             sg0, sg1, so0, so1):
    wid = lax.axis_index("s") * NC + lax.axis_index("c")
    ms = [m0, m1, m2, m3]
    wv = [w0, w1, w2, w3]
    rs = [r0, r1]
    os = [o0, o1]
    smi = [smi0, smi1, smi2, smi3]
    smw = [smw0, smw1, smw2, smw3]
    sg = [sg0, sg1]
    so = [so0, so1]
    ibase = wid * (GPW * 64 // 128)
    obase = wid * GPW

    def idx_src(s):
        s = jnp.minimum(s, STEPS - 1)
        return idx_hbm.at[pl.ds(pl.multiple_of(ibase + s * IDX_ROWS, 8),
                                IDX_ROWS)]

    def w_src(s):
        s = jnp.minimum(s, STEPS - 1)
        return w_hbm.at[pl.ds(pl.multiple_of(ibase + s * IDX_ROWS, 8),
                              IDX_ROWS)]

    def fire(mbuf, rbuf, sem):
        for j in range(IDX_ROWS):
            pltpu.async_copy(table.at[mbuf.at[j]],
                             rbuf.at[pl.ds(j * 128, 128)], sem)

    def compute(wbuf, rbuf, obuf):
        def hloop(h, c2):
            hb = h * 16
            for q2 in range(2):
                acc0 = jnp.zeros((16,), jnp.float32)
                acc1 = jnp.zeros((16,), jnp.float32)
                for c in range(NL):
                    wrow = jnp.full((16,), q2 * 4 + c, jnp.int32)
                    for lp in range(16):
                        s = plsc.load_gather(wbuf, [wrow,
                                                    jnp.full((16,), hb + lp,
                                                             jnp.int32)])
                        r = q2 * 512 + c * 128 + hb + lp
                        acc0 = acc0 + rbuf[r, pl.ds(0, 16)] * s
                        acc1 = acc1 + rbuf[r, pl.ds(16, 16)] * s
                g = q2 * 8 + h
                obuf[g, pl.ds(0, 16)] = acc0
                obuf[g, pl.ds(16, 16)] = acc1
            return c2

        lax.fori_loop(0, NH, hloop, 0)

    pltpu.async_copy(idx_src(0), ms[0], smi[0])
    pltpu.async_copy(w_src(0), wv[0], smw[0])
    pltpu.async_copy(idx_src(1), ms[1], smi[1])
    pltpu.async_copy(w_src(1), wv[1], smw[1])
    pltpu.make_async_copy(idx_src(0), ms[0], smi[0]).wait()
    fire(ms[0], rs[0], sg[0])

    def iter_t(t, carry):
        s0 = t * 4
        for k in range(4):
            s = s0 + k
            p = k % 2
            ka, kb = (k + 1) % 4, (k + 2) % 4
            pltpu.make_async_copy(idx_src(s + 1), ms[ka], smi[ka]).wait()
            fire(ms[ka], rs[1 - p], sg[1 - p])
            pltpu.async_copy(idx_src(s + 2), ms[kb], smi[kb])
            pltpu.async_copy(w_src(s + 2), wv[kb], smw[kb])
            pltpu.make_async_copy(table.at[pl.ds(0, RPS)], rs[p], sg[p]).wait()
            pltpu.make_async_copy(w_src(s), wv[k], smw[k]).wait()

            @pl.when(s >= 2)
            def _():
                pltpu.make_async_copy(os[p], out_hbm.at[pl.ds(0, G)],
                                      so[p]).wait()

            compute(wv[k], rs[p], os[p])
            pltpu.async_copy(
                os[p],
                out_hbm.at[pl.ds(pl.multiple_of(obase + s * G, 8), G)], so[p])
        return carry

    lax.fori_loop(0, STEPS // 4, iter_t, 0)
    pltpu.make_async_copy(table.at[pl.ds(0, RPS)], rs[0], sg[0]).wait()
    pltpu.make_async_copy(idx_src(0), ms[1], smi[1]).wait()
    pltpu.make_async_copy(w_src(0), wv[0], smw[0]).wait()
    pltpu.make_async_copy(w_src(0), wv[1], smw[1]).wait()
    pltpu.make_async_copy(os[0], out_hbm.at[pl.ds(0, G)], so[0]).wait()
    pltpu.make_async_copy(os[1], out_hbm.at[pl.ds(0, G)], so[1]).wait()


def kernel(query, reference_points, value, spatial_shapes, level_start_index,
           W_off, b_off, W_attn, b_attn, W_val, b_val, W_out, b_out):
    f32 = jnp.float32

    valproj = pl.pallas_call(
        _valproj_body,
        grid=(B * LV // QT,),
        in_specs=[
            pl.BlockSpec((QT, C), lambda i: (i, 0)),
            pl.BlockSpec((C, C), lambda i: (0, 0)),
            pl.BlockSpec((1, C), lambda i: (0, 0)),
        ],
        out_specs=pl.BlockSpec((QT, C), lambda i: (i, 0)),
        out_shape=jax.ShapeDtypeStruct((B * LV, C), f32),
    )(value.reshape(B * LV, C), W_val, b_val.reshape(1, C))
    table = valproj.reshape(B * LV * NH, D)

    woffp = W_off.reshape(C, NH, NL, NP, 2).transpose(0, 4, 1, 2, 3)
    woffp = woffp.reshape(C, C)
    boffp = b_off.reshape(NH, NL, NP, 2).transpose(3, 0, 1, 2).reshape(1, C)
    scale = jnp.asarray(_SS[:, ::-1].astype(np.float32))
    rp8 = (reference_points * scale).transpose(0, 1, 3, 2).reshape(B, LQ, 8)
    nrows = GROUPS * 64 // 128
    idx2d, w2d = pl.pallas_call(
        _sample_body,
        grid=(B, NQT),
        in_specs=[
            pl.BlockSpec((1, QT, C), lambda b, i: (b, i, 0)),
            pl.BlockSpec((1, QT, 8), lambda b, i: (b, i, 0)),
            pl.BlockSpec((C, C), lambda b, i: (0, 0)),
            pl.BlockSpec((1, C), lambda b, i: (0, 0)),
            pl.BlockSpec((C, NH * 16), lambda b, i: (0, 0)),
            pl.BlockSpec((1, NH * 16), lambda b, i: (0, 0)),
        ],
        out_specs=[
            pl.BlockSpec((QT * 4, 128), lambda b, i: (b * NQT + i, 0)),
            pl.BlockSpec((QT * 4, 128), lambda b, i: (b * NQT + i, 0)),
        ],
        out_shape=[
            jax.ShapeDtypeStruct((nrows, 128), jnp.int32),
            jax.ShapeDtypeStruct((nrows, 128), f32),
        ],
    )(query, rp8, woffp, boffp, W_attn, b_attn.reshape(1, NH * 16))

    mesh = plsc.VectorSubcoreMesh(core_axis_name="c", subcore_axis_name="s",
                                  num_cores=NC, num_subcores=NS)
    sc = pl.kernel(
        _sc_body,
        out_type=jax.ShapeDtypeStruct((GROUPS, D), f32),
        mesh=mesh,
        compiler_params=pltpu.CompilerParams(needs_layout_passes=False,
                                             use_tc_tiling_on_sc=False),
        scratch_types=(
            [pltpu.VMEM((IDX_ROWS, 128), jnp.int32)] * 4
            + [pltpu.VMEM((IDX_ROWS, 128), f32)] * 4
            + [pltpu.VMEM((RPS, D), f32)] * 2
            + [pltpu.VMEM((G, D), f32)] * 2
            + [pltpu.SemaphoreType.DMA] * 12
        ),
    )
    sampled = sc(table, idx2d, w2d)

    out = pl.pallas_call(
        _outproj_body,
        grid=(B * LQ // QT,),
        in_specs=[
            pl.BlockSpec((QT, C), lambda i: (i, 0)),
            pl.BlockSpec((C, C), lambda i: (0, 0)),
            pl.BlockSpec((1, C), lambda i: (0, 0)),
        ],
        out_specs=pl.BlockSpec((QT, C), lambda i: (i, 0)),
        out_shape=jax.ShapeDtypeStruct((B * LQ, C), f32),
    )(sampled.reshape(B * LQ, C), W_out, b_out.reshape(1, C))
    return out.reshape(B, LQ, C)


# speedup vs baseline: 183.6725x; 1.1927x over previous
import functools

import jax
import jax.numpy as jnp
import numpy as np
from jax import lax
from jax.experimental import pallas as pl
from jax.experimental.pallas import tpu as pltpu
from jax.experimental.pallas import tpu_sc as plsc

_SS = np.array([[64, 64], [32, 32], [16, 16], [8, 8]], dtype=np.int64)
_AREAS = _SS[:, 0] * _SS[:, 1]
_LSI = np.concatenate([np.zeros(1, dtype=np.int64), np.cumsum(_AREAS)[:-1]])
LV = int(_AREAS.sum())
LQ = LV
B = 4
C = 256
NH, NL, NP = 8, 4, 4
D = C // NH

QT = 680
NQT = LQ // QT

NC, NS = 2, 16
NW = NC * NS
GROUPS = B * LQ * NH
GPW = GROUPS // NW
G = 16
STEPS = GPW // G
RPS = G * NL * NP * 4
IDX_ROWS = RPS // 128

def _valproj_body(v_ref, w_ref, b_ref, o_ref):
    o_ref[...] = (jnp.dot(v_ref[...], w_ref[...],
                          preferred_element_type=jnp.float32)
                  + b_ref[...]).astype(jnp.bfloat16)


def _outproj_body(x_ref, w_ref, b_ref, o_ref):
    o_ref[...] = jnp.dot(x_ref[...], w_ref[...],
                         preferred_element_type=jnp.float32) + b_ref[...]


def _sample_body(q_ref, rp_ref, woff_ref, boff_ref, wattn_ref, battn_ref,
                 idx_ref, w_ref):
    i32, f32 = jnp.int32, jnp.float32
    b = pl.program_id(0)
    q = q_ref[0]
    off = jnp.dot(q, woff_ref[...],
                  preferred_element_type=f32) + boff_ref[...]
    attn = jnp.dot(q, wattn_ref[...],
                   preferred_element_type=f32) + battn_ref[...]
    m = jnp.max(attn, axis=1, keepdims=True)
    e = jnp.exp(attn - m)
    r0 = lax.broadcasted_iota(i32, (128, 128), 0)
    c0 = lax.broadcasted_iota(i32, (128, 128), 1)
    seg = (r0 // 16 == c0 // 16).astype(f32)
    denom = jnp.dot(e, seg, preferred_element_type=f32)
    aw = e / denom
    rp8 = rp_ref[0]
    kio = lax.broadcasted_iota(i32, (1, 128), 1)
    lidx = (kio % 16) // 4
    h_lane = kio // 16
    wrow_i = lax.shift_right_logical(jnp.full((1, 128), 64, i32), lidx)
    wrow = wrow_i.astype(f32)
    base_c = jnp.where(
        lidx == 0, 0,
        jnp.where(lidx == 1, int(_LSI[1]) * NH,
                  jnp.where(lidx == 2, int(_LSI[2]) * NH,
                            int(_LSI[3]) * NH))) + h_lane
    xb = jnp.zeros((QT, 128), f32)
    yb = jnp.zeros((QT, 128), f32)
    for l in range(NL):
        msk = (lidx == l).astype(f32)
        xb = xb + msk * rp8[:, l:l + 1]
        yb = yb + msk * rp8[:, NL + l:NL + l + 1]
    x = xb + off[:, :128] - 0.5
    y = yb + off[:, 128:] - 0.5
    x0 = jnp.floor(x)
    y0 = jnp.floor(y)
    fx = x - x0
    fy = y - y0
    vx0 = (x0 >= 0.0) & (x0 <= wrow - 1.0)
    vx1 = (x0 + 1.0 >= 0.0) & (x0 + 1.0 <= wrow - 1.0)
    vy0 = (y0 >= 0.0) & (y0 <= wrow - 1.0)
    vy1 = (y0 + 1.0 >= 0.0) & (y0 + 1.0 <= wrow - 1.0)
    xc0 = jnp.clip(x0, 0.0, wrow - 1.0).astype(i32)
    xc1 = jnp.clip(x0 + 1.0, 0.0, wrow - 1.0).astype(i32)
    yc0 = jnp.clip(y0, 0.0, wrow - 1.0).astype(i32)
    yc1 = jnp.clip(y0 + 1.0, 0.0, wrow - 1.0).astype(i32)
    base = base_c + b * (LQ * NH)
    i00 = base + (yc0 * wrow_i + xc0) * NH
    i10 = base + (yc0 * wrow_i + xc1) * NH
    i01 = base + (yc1 * wrow_i + xc0) * NH
    i11 = base + (yc1 * wrow_i + xc1) * NH
    w00 = (1.0 - fx) * (1.0 - fy) * aw * (vx0 & vy0).astype(f32)
    w10 = fx * (1.0 - fy) * aw * (vx1 & vy0).astype(f32)
    w01 = (1.0 - fx) * fy * aw * (vx0 & vy1).astype(f32)
    w11 = fx * fy * aw * (vx1 & vy1).astype(f32)
    icat = jnp.concatenate([t.reshape(QT, 1, 128) for t in
                            (i00, i10, i01, i11)], axis=1)
    wcat = jnp.concatenate([t.reshape(QT, 1, 128) for t in
                            (w00, w10, w01, w11)], axis=1)
    idx_ref[...] = icat.reshape(QT * 4, 128)
    w_ref[...] = wcat.reshape(QT * 4, 128)


def _sc_body(table, idx_hbm, w_hbm, out_hbm,
             m0, m1, m2, m3, w0, w1, w2, w3, r0, r1, o0, o1,
             smi0, smi1, smi2, smi3, smw0, smw1, smw2, smw3,
             sg0, sg1, so0, so1):
    wid = lax.axis_index("s") * NC + lax.axis_index("c")
    ms = [m0, m1, m2, m3]
    wv = [w0, w1, w2, w3]
    rs = [r0, r1]
    os = [o0, o1]
    smi = [smi0, smi1, smi2, smi3]
    smw = [smw0, smw1, smw2, smw3]
    sg = [sg0, sg1]
    so = [so0, so1]
    ibase = wid * (GPW * 64 // 128)
    obase = wid * GPW

    def idx_src(s):
        s = jnp.minimum(s, STEPS - 1)
        return idx_hbm.at[pl.ds(pl.multiple_of(ibase + s * IDX_ROWS, 8),
                                IDX_ROWS)]

    def w_src(s):
        s = jnp.minimum(s, STEPS - 1)
        return w_hbm.at[pl.ds(pl.multiple_of(ibase + s * IDX_ROWS, 8),
                              IDX_ROWS)]

    def fire(mbuf, rbuf, sem):
        for j in range(IDX_ROWS):
            pltpu.async_copy(table.at[mbuf.at[j]],
                             rbuf.at[pl.ds(j * 128, 128)], sem)

    def compute(wbuf, rbuf, obuf):
        def hloop(h, c2):
            hb = h * 16
            for q2 in range(2):
                acc0 = jnp.zeros((16,), jnp.float32)
                acc1 = jnp.zeros((16,), jnp.float32)
                for c in range(NL):
                    wrow = jnp.full((16,), q2 * 4 + c, jnp.int32)
                    for lp in range(16):
                        s = plsc.load_gather(wbuf, [wrow,
                                                    jnp.full((16,), hb + lp,
                                                             jnp.int32)])
                        r = q2 * 512 + c * 128 + hb + lp
                        ev, od = plsc.unpack(rbuf[r],
                                             format=plsc.PackFormat.INTERLEAVED)
                        acc0 = acc0 + ev * s
                        acc1 = acc1 + od * s
                g = q2 * 8 + h
                obuf[g, pl.ds(0, 16)] = acc0
                obuf[g, pl.ds(16, 16)] = acc1
            return c2

        lax.fori_loop(0, NH, hloop, 0)

    pltpu.async_copy(idx_src(0), ms[0], smi[0])
    pltpu.async_copy(w_src(0), wv[0], smw[0])
    pltpu.async_copy(idx_src(1), ms[1], smi[1])
    pltpu.async_copy(w_src(1), wv[1], smw[1])
    pltpu.make_async_copy(idx_src(0), ms[0], smi[0]).wait()
    fire(ms[0], rs[0], sg[0])

    def iter_t(t, carry):
        s0 = t * 4
        for k in range(4):
            s = s0 + k
            p = k % 2
            ka, kb = (k + 1) % 4, (k + 2) % 4
            pltpu.make_async_copy(idx_src(s + 1), ms[ka], smi[ka]).wait()
            fire(ms[ka], rs[1 - p], sg[1 - p])
            pltpu.async_copy(idx_src(s + 2), ms[kb], smi[kb])
            pltpu.async_copy(w_src(s + 2), wv[kb], smw[kb])
            pltpu.make_async_copy(table.at[pl.ds(0, RPS)], rs[p], sg[p]).wait()
            pltpu.make_async_copy(w_src(s), wv[k], smw[k]).wait()

            @pl.when(s >= 2)
            def _():
                pltpu.make_async_copy(os[p], out_hbm.at[pl.ds(0, G)],
                                      so[p]).wait()

            compute(wv[k], rs[p], os[p])
            pltpu.async_copy(
                os[p],
                out_hbm.at[pl.ds(pl.multiple_of(obase + s * G, 8), G)], so[p])
        return carry

    lax.fori_loop(0, STEPS // 4, iter_t, 0)
    pltpu.make_async_copy(table.at[pl.ds(0, RPS)], rs[0], sg[0]).wait()
    pltpu.make_async_copy(idx_src(0), ms[1], smi[1]).wait()
    pltpu.make_async_copy(w_src(0), wv[0], smw[0]).wait()
    pltpu.make_async_copy(w_src(0), wv[1], smw[1]).wait()
    pltpu.make_async_copy(os[0], out_hbm.at[pl.ds(0, G)], so[0]).wait()
    pltpu.make_async_copy(os[1], out_hbm.at[pl.ds(0, G)], so[1]).wait()


def kernel(query, reference_points, value, spatial_shapes, level_start_index,
           W_off, b_off, W_attn, b_attn, W_val, b_val, W_out, b_out):
    f32 = jnp.float32

    VT = 640
    valproj = pl.pallas_call(
        _valproj_body,
        grid=(B * LV // VT,),
        in_specs=[
            pl.BlockSpec((VT, C), lambda i: (i, 0)),
            pl.BlockSpec((C, C), lambda i: (0, 0)),
            pl.BlockSpec((1, C), lambda i: (0, 0)),
        ],
        out_specs=pl.BlockSpec((VT, C), lambda i: (i, 0)),
        out_shape=jax.ShapeDtypeStruct((B * LV, C), jnp.bfloat16),
    )(value.reshape(B * LV, C), W_val, b_val.reshape(1, C))
    table = valproj.reshape(B * LV * NH, D)

    woffp = W_off.reshape(C, NH, NL, NP, 2).transpose(0, 4, 1, 2, 3)
    woffp = woffp.reshape(C, C)
    boffp = b_off.reshape(NH, NL, NP, 2).transpose(3, 0, 1, 2).reshape(1, C)
    scale = jnp.asarray(_SS[:, ::-1].astype(np.float32))
    rp8 = (reference_points * scale).transpose(0, 1, 3, 2).reshape(B, LQ, 8)
    nrows = GROUPS * 64 // 128
    idx2d, w2d = pl.pallas_call(
        _sample_body,
        grid=(B, NQT),
        in_specs=[
            pl.BlockSpec((1, QT, C), lambda b, i: (b, i, 0)),
            pl.BlockSpec((1, QT, 8), lambda b, i: (b, i, 0)),
            pl.BlockSpec((C, C), lambda b, i: (0, 0)),
            pl.BlockSpec((1, C), lambda b, i: (0, 0)),
            pl.BlockSpec((C, NH * 16), lambda b, i: (0, 0)),
            pl.BlockSpec((1, NH * 16), lambda b, i: (0, 0)),
        ],
        out_specs=[
            pl.BlockSpec((QT * 4, 128), lambda b, i: (b * NQT + i, 0)),
            pl.BlockSpec((QT * 4, 128), lambda b, i: (b * NQT + i, 0)),
        ],
        out_shape=[
            jax.ShapeDtypeStruct((nrows, 128), jnp.int32),
            jax.ShapeDtypeStruct((nrows, 128), f32),
        ],
    )(query, rp8, woffp, boffp, W_attn, b_attn.reshape(1, NH * 16))

    mesh = plsc.VectorSubcoreMesh(core_axis_name="c", subcore_axis_name="s",
                                  num_cores=NC, num_subcores=NS)
    sc = pl.kernel(
        _sc_body,
        out_type=jax.ShapeDtypeStruct((GROUPS, D), f32),
        mesh=mesh,
        compiler_params=pltpu.CompilerParams(needs_layout_passes=False,
                                             use_tc_tiling_on_sc=False),
        scratch_types=(
            [pltpu.VMEM((IDX_ROWS, 128), jnp.int32)] * 4
            + [pltpu.VMEM((IDX_ROWS, 128), f32)] * 4
            + [pltpu.VMEM((RPS, D), jnp.bfloat16)] * 2
            + [pltpu.VMEM((G, D), f32)] * 2
            + [pltpu.SemaphoreType.DMA] * 12
        ),
    )
    sampled = sc(table, idx2d, w2d)

    perm = np.concatenate([np.arange(0, D, 2), np.arange(1, D, 2)])
    W_outp = W_out.reshape(NH, D, C)[:, perm, :].reshape(C, C)
    out = pl.pallas_call(
        _outproj_body,
        grid=(B * LQ // QT,),
        in_specs=[
            pl.BlockSpec((QT, C), lambda i: (i, 0)),
            pl.BlockSpec((C, C), lambda i: (0, 0)),
            pl.BlockSpec((1, C), lambda i: (0, 0)),
        ],
        out_specs=pl.BlockSpec((QT, C), lambda i: (i, 0)),
        out_shape=jax.ShapeDtypeStruct((B * LQ, C), f32),
    )(sampled.reshape(B * LQ, C), W_outp, b_out.reshape(1, C))
    return out.reshape(B, LQ, C)


# speedup vs baseline: 188.7191x; 1.0275x over previous
import functools

import jax
import jax.numpy as jnp
import numpy as np
from jax import lax
from jax.experimental import pallas as pl
from jax.experimental.pallas import tpu as pltpu
from jax.experimental.pallas import tpu_sc as plsc

_SS = np.array([[64, 64], [32, 32], [16, 16], [8, 8]], dtype=np.int64)
_AREAS = _SS[:, 0] * _SS[:, 1]
_LSI = np.concatenate([np.zeros(1, dtype=np.int64), np.cumsum(_AREAS)[:-1]])
LV = int(_AREAS.sum())
LQ = LV
B = 4
C = 256
NH, NL, NP = 8, 4, 4
D = C // NH

QT = 680
NQT = LQ // QT

NC, NS = 2, 16
NW = NC * NS
GROUPS = B * LQ * NH
GPW = GROUPS // NW
G = 16
STEPS = GPW // G
RPS = G * NL * NP * 4
IDX_ROWS = RPS // 128

def _valproj_body(v_ref, w_ref, b_ref, o_ref):
    o_ref[...] = (jnp.dot(v_ref[...], w_ref[...],
                          preferred_element_type=jnp.float32)
                  + b_ref[...]).astype(jnp.bfloat16)


def _outproj_body(x_ref, w_ref, b_ref, o_ref):
    o_ref[...] = jnp.dot(x_ref[...], w_ref[...],
                         preferred_element_type=jnp.float32) + b_ref[...]


def _sample_body(q_ref, rp_ref, woff_ref, boff_ref, wattn_ref, battn_ref,
                 idx_ref, w_ref):
    i32, f32 = jnp.int32, jnp.float32
    b = pl.program_id(0)
    q = q_ref[0]
    off = jnp.dot(q, woff_ref[...],
                  preferred_element_type=f32) + boff_ref[...]
    attn = jnp.dot(q, wattn_ref[...],
                   preferred_element_type=f32) + battn_ref[...]
    m = jnp.max(attn, axis=1, keepdims=True)
    e = jnp.exp(attn - m)
    r0 = lax.broadcasted_iota(i32, (128, 128), 0)
    c0 = lax.broadcasted_iota(i32, (128, 128), 1)
    seg = (r0 // 16 == c0 // 16).astype(f32)
    denom = jnp.dot(e, seg, preferred_element_type=f32)
    aw = e / denom
    rp8 = rp_ref[0]
    kio = lax.broadcasted_iota(i32, (1, 128), 1)
    lidx = (kio % 16) // 4
    h_lane = kio // 16
    wrow_i = lax.shift_right_logical(jnp.full((1, 128), 64, i32), lidx)
    wrow = wrow_i.astype(f32)
    base_c = jnp.where(
        lidx == 0, 0,
        jnp.where(lidx == 1, int(_LSI[1]) * NH,
                  jnp.where(lidx == 2, int(_LSI[2]) * NH,
                            int(_LSI[3]) * NH))) + h_lane
    xb = jnp.zeros((QT, 128), f32)
    yb = jnp.zeros((QT, 128), f32)
    for l in range(NL):
        msk = (lidx == l).astype(f32)
        xb = xb + msk * rp8[:, l:l + 1]
        yb = yb + msk * rp8[:, NL + l:NL + l + 1]
    x = xb + off[:, :128] - 0.5
    y = yb + off[:, 128:] - 0.5
    x0 = jnp.floor(x)
    y0 = jnp.floor(y)
    fx = x - x0
    fy = y - y0
    vx0 = (x0 >= 0.0) & (x0 <= wrow - 1.0)
    vx1 = (x0 + 1.0 >= 0.0) & (x0 + 1.0 <= wrow - 1.0)
    vy0 = (y0 >= 0.0) & (y0 <= wrow - 1.0)
    vy1 = (y0 + 1.0 >= 0.0) & (y0 + 1.0 <= wrow - 1.0)
    xc0 = jnp.clip(x0, 0.0, wrow - 1.0).astype(i32)
    xc1 = jnp.clip(x0 + 1.0, 0.0, wrow - 1.0).astype(i32)
    yc0 = jnp.clip(y0, 0.0, wrow - 1.0).astype(i32)
    yc1 = jnp.clip(y0 + 1.0, 0.0, wrow - 1.0).astype(i32)
    base = base_c + b * (LQ * NH)
    i00 = base + (yc0 * wrow_i + xc0) * NH
    dx = (xc1 - xc0) * NH
    dy = (yc1 - yc0) * (wrow_i * NH)
    i10 = i00 + dx
    i01 = i00 + dy
    i11 = i01 + dx
    ax0 = (1.0 - fx) * vx0.astype(f32) * aw
    ax1 = fx * vx1.astype(f32) * aw
    by0 = (1.0 - fy) * vy0.astype(f32)
    by1 = fy * vy1.astype(f32)
    w00 = ax0 * by0
    w10 = ax1 * by0
    w01 = ax0 * by1
    w11 = ax1 * by1
    icat = jnp.concatenate([t.reshape(QT, 1, 128) for t in
                            (i00, i10, i01, i11)], axis=1)
    wcat = jnp.concatenate([t.reshape(QT, 1, 128) for t in
                            (w00, w10, w01, w11)], axis=1)
    idx_ref[...] = icat.reshape(QT * 4, 128)
    w_ref[...] = wcat.reshape(QT * 4, 128)


def _sc_body(table, idx_hbm, w_hbm, out_hbm,
             m0, m1, m2, m3, w0, w1, w2, w3, r0, r1, o0, o1,
             smi0, smi1, smi2, smi3, smw0, smw1, smw2, smw3,
             sg0, sg1, so0, so1):
    wid = lax.axis_index("s") * NC + lax.axis_index("c")
    ms = [m0, m1, m2, m3]
    wv = [w0, w1, w2, w3]
    rs = [r0, r1]
    os = [o0, o1]
    smi = [smi0, smi1, smi2, smi3]
    smw = [smw0, smw1, smw2, smw3]
    sg = [sg0, sg1]
    so = [so0, so1]
    ibase = wid * (GPW * 64 // 128)
    obase = wid * GPW

    def idx_src(s):
        s = jnp.minimum(s, STEPS - 1)
        return idx_hbm.at[pl.ds(pl.multiple_of(ibase + s * IDX_ROWS, 8),
                                IDX_ROWS)]

    def w_src(s):
        s = jnp.minimum(s, STEPS - 1)
        return w_hbm.at[pl.ds(pl.multiple_of(ibase + s * IDX_ROWS, 8),
                              IDX_ROWS)]

    def fire(mbuf, rbuf, sem):
        for j in range(IDX_ROWS):
            pltpu.async_copy(table.at[mbuf.at[j]],
                             rbuf.at[pl.ds(j * 128, 128)], sem)

    gdn = lax.GatherDimensionNumbers(offset_dims=(), collapsed_slice_dims=(0,),
                                     start_index_map=(0,))

    def splat(v, j):
        idx = jnp.full((16, 1), j, jnp.int32)
        return lax.gather(v, idx, gdn, (1,),
                          mode=lax.GatherScatterMode.PROMISE_IN_BOUNDS)

    def compute(wbuf, rbuf, obuf):
        def hloop(h, c2):
            hb = h * 16
            for q2 in range(2):
                acc0 = jnp.zeros((16,), jnp.float32)
                acc1 = jnp.zeros((16,), jnp.float32)
                for c in range(NL):
                    w16 = wbuf[q2 * 4 + c, pl.ds(hb, 16)]
                    for lp in range(16):
                        s = splat(w16, lp)
                        r = q2 * 512 + c * 128 + hb + lp
                        ev, od = plsc.unpack(rbuf[r],
                                             format=plsc.PackFormat.INTERLEAVED)
                        acc0 = acc0 + ev * s
                        acc1 = acc1 + od * s
                g = q2 * 8 + h
                obuf[g, pl.ds(0, 16)] = acc0
                obuf[g, pl.ds(16, 16)] = acc1
            return c2

        lax.fori_loop(0, NH, hloop, 0)

    pltpu.async_copy(idx_src(0), ms[0], smi[0])
    pltpu.async_copy(w_src(0), wv[0], smw[0])
    pltpu.async_copy(idx_src(1), ms[1], smi[1])
    pltpu.async_copy(w_src(1), wv[1], smw[1])
    pltpu.make_async_copy(idx_src(0), ms[0], smi[0]).wait()
    fire(ms[0], rs[0], sg[0])

    def iter_t(t, carry):
        s0 = t * 4
        for k in range(4):
            s = s0 + k
            p = k % 2
            ka, kb = (k + 1) % 4, (k + 2) % 4
            pltpu.make_async_copy(idx_src(s + 1), ms[ka], smi[ka]).wait()
            fire(ms[ka], rs[1 - p], sg[1 - p])
            pltpu.async_copy(idx_src(s + 2), ms[kb], smi[kb])
            pltpu.async_copy(w_src(s + 2), wv[kb], smw[kb])
            pltpu.make_async_copy(table.at[pl.ds(0, RPS)], rs[p], sg[p]).wait()
            pltpu.make_async_copy(w_src(s), wv[k], smw[k]).wait()

            @pl.when(s >= 2)
            def _():
                pltpu.make_async_copy(os[p], out_hbm.at[pl.ds(0, G)],
                                      so[p]).wait()

            compute(wv[k], rs[p], os[p])
            pltpu.async_copy(
                os[p],
                out_hbm.at[pl.ds(pl.multiple_of(obase + s * G, 8), G)], so[p])
        return carry

    lax.fori_loop(0, STEPS // 4, iter_t, 0)
    pltpu.make_async_copy(table.at[pl.ds(0, RPS)], rs[0], sg[0]).wait()
    pltpu.make_async_copy(idx_src(0), ms[1], smi[1]).wait()
    pltpu.make_async_copy(w_src(0), wv[0], smw[0]).wait()
    pltpu.make_async_copy(w_src(0), wv[1], smw[1]).wait()
    pltpu.make_async_copy(os[0], out_hbm.at[pl.ds(0, G)], so[0]).wait()
    pltpu.make_async_copy(os[1], out_hbm.at[pl.ds(0, G)], so[1]).wait()


def kernel(query, reference_points, value, spatial_shapes, level_start_index,
           W_off, b_off, W_attn, b_attn, W_val, b_val, W_out, b_out):
    f32 = jnp.float32

    VT = 640
    valproj = pl.pallas_call(
        _valproj_body,
        grid=(B * LV // VT,),
        in_specs=[
            pl.BlockSpec((VT, C), lambda i: (i, 0)),
            pl.BlockSpec((C, C), lambda i: (0, 0)),
            pl.BlockSpec((1, C), lambda i: (0, 0)),
        ],
        out_specs=pl.BlockSpec((VT, C), lambda i: (i, 0)),
        out_shape=jax.ShapeDtypeStruct((B * LV, C), jnp.bfloat16),
    )(value.reshape(B * LV, C), W_val, b_val.reshape(1, C))
    table = valproj.reshape(B * LV * NH, D)

    woffp = W_off.reshape(C, NH, NL, NP, 2).transpose(0, 4, 1, 2, 3)
    woffp = woffp.reshape(C, C)
    boffp = b_off.reshape(NH, NL, NP, 2).transpose(3, 0, 1, 2).reshape(1, C)
    scale = jnp.asarray(_SS[:, ::-1].astype(np.float32))
    rp8 = (reference_points * scale).transpose(0, 1, 3, 2).reshape(B, LQ, 8)
    nrows = GROUPS * 64 // 128
    idx2d, w2d = pl.pallas_call(
        _sample_body,
        grid=(B, NQT),
        in_specs=[
            pl.BlockSpec((1, QT, C), lambda b, i: (b, i, 0)),
            pl.BlockSpec((1, QT, 8), lambda b, i: (b, i, 0)),
            pl.BlockSpec((C, C), lambda b, i: (0, 0)),
            pl.BlockSpec((1, C), lambda b, i: (0, 0)),
            pl.BlockSpec((C, NH * 16), lambda b, i: (0, 0)),
            pl.BlockSpec((1, NH * 16), lambda b, i: (0, 0)),
        ],
        out_specs=[
            pl.BlockSpec((QT * 4, 128), lambda b, i: (b * NQT + i, 0)),
            pl.BlockSpec((QT * 4, 128), lambda b, i: (b * NQT + i, 0)),
        ],
        out_shape=[
            jax.ShapeDtypeStruct((nrows, 128), jnp.int32),
            jax.ShapeDtypeStruct((nrows, 128), f32),
        ],
    )(query, rp8, woffp, boffp, W_attn, b_attn.reshape(1, NH * 16))

    mesh = plsc.VectorSubcoreMesh(core_axis_name="c", subcore_axis_name="s",
                                  num_cores=NC, num_subcores=NS)
    sc = pl.kernel(
        _sc_body,
        out_type=jax.ShapeDtypeStruct((GROUPS, D), f32),
        mesh=mesh,
        compiler_params=pltpu.CompilerParams(needs_layout_passes=False,
                                             use_tc_tiling_on_sc=False),
        scratch_types=(
            [pltpu.VMEM((IDX_ROWS, 128), jnp.int32)] * 4
            + [pltpu.VMEM((IDX_ROWS, 128), f32)] * 4
            + [pltpu.VMEM((RPS, D), jnp.bfloat16)] * 2
            + [pltpu.VMEM((G, D), f32)] * 2
            + [pltpu.SemaphoreType.DMA] * 12
        ),
    )
    sampled = sc(table, idx2d, w2d)

    perm = np.concatenate([np.arange(0, D, 2), np.arange(1, D, 2)])
    W_outp = W_out.reshape(NH, D, C)[:, perm, :].reshape(C, C)
    out = pl.pallas_call(
        _outproj_body,
        grid=(B * LQ // QT,),
        in_specs=[
            pl.BlockSpec((QT, C), lambda i: (i, 0)),
            pl.BlockSpec((C, C), lambda i: (0, 0)),
            pl.BlockSpec((1, C), lambda i: (0, 0)),
        ],
        out_specs=pl.BlockSpec((QT, C), lambda i: (i, 0)),
        out_shape=jax.ShapeDtypeStruct((B * LQ, C), f32),
    )(sampled.reshape(B * LQ, C), W_outp, b_out.reshape(1, C))
    return out.reshape(B, LQ, C)


# speedup vs baseline: 188.9902x; 1.0014x over previous
import functools

import jax
import jax.numpy as jnp
import numpy as np
from jax import lax
from jax.experimental import pallas as pl
from jax.experimental.pallas import tpu as pltpu
from jax.experimental.pallas import tpu_sc as plsc

_SS = np.array([[64, 64], [32, 32], [16, 16], [8, 8]], dtype=np.int64)
_AREAS = _SS[:, 0] * _SS[:, 1]
_LSI = np.concatenate([np.zeros(1, dtype=np.int64), np.cumsum(_AREAS)[:-1]])
LV = int(_AREAS.sum())
LQ = LV
B = 4
C = 256
NH, NL, NP = 8, 4, 4
D = C // NH

QT = 680
NQT = LQ // QT

NC, NS = 2, 16
NW = NC * NS
GROUPS = B * LQ * NH
GPW = GROUPS // NW
G = 16
STEPS = GPW // G
RPS = G * NL * NP * 4
IDX_ROWS = RPS // 128

def _valproj_body(v_ref, w_ref, b_ref, o_ref):
    o_ref[...] = (jnp.dot(v_ref[...].astype(jnp.bfloat16),
                          w_ref[...].astype(jnp.bfloat16),
                          preferred_element_type=jnp.float32)
                  + b_ref[...]).astype(jnp.bfloat16)


def _outproj_body(x_ref, w_ref, b_ref, o_ref):
    o_ref[...] = jnp.dot(x_ref[...].astype(jnp.bfloat16),
                         w_ref[...].astype(jnp.bfloat16),
                         preferred_element_type=jnp.float32) + b_ref[...]


def _sample_body(q_ref, rp_ref, woff_ref, boff_ref, wattn_ref, battn_ref,
                 idx_ref, w_ref):
    i32, f32 = jnp.int32, jnp.float32
    b = pl.program_id(0)
    q = q_ref[0]
    off = jnp.dot(q, woff_ref[...],
                  preferred_element_type=f32) + boff_ref[...]
    attn = jnp.dot(q, wattn_ref[...],
                   preferred_element_type=f32) + battn_ref[...]
    m = jnp.max(attn, axis=1, keepdims=True)
    e = jnp.exp(attn - m)
    r0 = lax.broadcasted_iota(i32, (128, 128), 0)
    c0 = lax.broadcasted_iota(i32, (128, 128), 1)
    seg = (r0 // 16 == c0 // 16).astype(f32)
    denom = jnp.dot(e, seg, preferred_element_type=f32)
    aw = e / denom
    rp8 = rp_ref[0]
    kio = lax.broadcasted_iota(i32, (1, 128), 1)
    lidx = (kio % 16) // 4
    h_lane = kio // 16
    wrow_i = lax.shift_right_logical(jnp.full((1, 128), 64, i32), lidx)
    wrow = wrow_i.astype(f32)
    base_c = jnp.where(
        lidx == 0, 0,
        jnp.where(lidx == 1, int(_LSI[1]) * NH,
                  jnp.where(lidx == 2, int(_LSI[2]) * NH,
                            int(_LSI[3]) * NH))) + h_lane
    xb = jnp.zeros((QT, 128), f32)
    yb = jnp.zeros((QT, 128), f32)
    for l in range(NL):
        msk = (lidx == l).astype(f32)
        xb = xb + msk * rp8[:, l:l + 1]
        yb = yb + msk * rp8[:, NL + l:NL + l + 1]
    x = xb + off[:, :128] - 0.5
    y = yb + off[:, 128:] - 0.5
    x0 = jnp.floor(x)
    y0 = jnp.floor(y)
    fx = x - x0
    fy = y - y0
    vx0 = (x0 >= 0.0) & (x0 <= wrow - 1.0)
    vx1 = (x0 + 1.0 >= 0.0) & (x0 + 1.0 <= wrow - 1.0)
    vy0 = (y0 >= 0.0) & (y0 <= wrow - 1.0)
    vy1 = (y0 + 1.0 >= 0.0) & (y0 + 1.0 <= wrow - 1.0)
    xc0 = jnp.clip(x0, 0.0, wrow - 1.0).astype(i32)
    xc1 = jnp.clip(x0 + 1.0, 0.0, wrow - 1.0).astype(i32)
    yc0 = jnp.clip(y0, 0.0, wrow - 1.0).astype(i32)
    yc1 = jnp.clip(y0 + 1.0, 0.0, wrow - 1.0).astype(i32)
    base = base_c + b * (LQ * NH)
    i00 = base + (yc0 * wrow_i + xc0) * NH
    dx = (xc1 - xc0) * NH
    dy = (yc1 - yc0) * (wrow_i * NH)
    i10 = i00 + dx
    i01 = i00 + dy
    i11 = i01 + dx
    ax0 = (1.0 - fx) * vx0.astype(f32) * aw
    ax1 = fx * vx1.astype(f32) * aw
    by0 = (1.0 - fy) * vy0.astype(f32)
    by1 = fy * vy1.astype(f32)
    w00 = ax0 * by0
    w10 = ax1 * by0
    w01 = ax0 * by1
    w11 = ax1 * by1
    icat = jnp.concatenate([t.reshape(QT, 1, 128) for t in
                            (i00, i10, i01, i11)], axis=1)
    wcat = jnp.concatenate([t.reshape(QT, 1, 128) for t in
                            (w00, w10, w01, w11)], axis=1)
    idx_ref[...] = icat.reshape(QT * 4, 128)
    w_ref[...] = wcat.reshape(QT * 4, 128)


def _sc_body(table, idx_hbm, w_hbm, out_hbm,
             m0, m1, m2, m3, w0, w1, w2, w3, r0, r1, o0, o1,
             smi0, smi1, smi2, smi3, smw0, smw1, smw2, smw3,
             sg0, sg1, so0, so1):
    wid = lax.axis_index("s") * NC + lax.axis_index("c")
    ms = [m0, m1, m2, m3]
    wv = [w0, w1, w2, w3]
    rs = [r0, r1]
    os = [o0, o1]
    smi = [smi0, smi1, smi2, smi3]
    smw = [smw0, smw1, smw2, smw3]
    sg = [sg0, sg1]
    so = [so0, so1]
    ibase = wid * (GPW * 64 // 128)
    obase = wid * GPW

    def idx_src(s):
        s = jnp.minimum(s, STEPS - 1)
        return idx_hbm.at[pl.ds(pl.multiple_of(ibase + s * IDX_ROWS, 8),
                                IDX_ROWS)]

    def w_src(s):
        s = jnp.minimum(s, STEPS - 1)
        return w_hbm.at[pl.ds(pl.multiple_of(ibase + s * IDX_ROWS, 8),
                              IDX_ROWS)]

    def fire(mbuf, rbuf, sem):
        for j in range(IDX_ROWS):
            pltpu.async_copy(table.at[mbuf.at[j]],
                             rbuf.at[pl.ds(j * 128, 128)], sem)

    gdn = lax.GatherDimensionNumbers(offset_dims=(), collapsed_slice_dims=(0,),
                                     start_index_map=(0,))

    def splat(v, j):
        idx = jnp.full((16, 1), j, jnp.int32)
        return lax.gather(v, idx, gdn, (1,),
                          mode=lax.GatherScatterMode.PROMISE_IN_BOUNDS)

    def compute(wbuf, rbuf, obuf):
        def hloop(h, c2):
            hb = h * 16
            for q2 in range(2):
                acc0 = jnp.zeros((16,), jnp.float32)
                acc1 = jnp.zeros((16,), jnp.float32)
                for c in range(NL):
                    w16 = wbuf[q2 * 4 + c, pl.ds(hb, 16)]
                    for lp in range(16):
                        s = splat(w16, lp)
                        r = q2 * 512 + c * 128 + hb + lp
                        ev, od = plsc.unpack(rbuf[r],
                                             format=plsc.PackFormat.INTERLEAVED)
                        acc0 = acc0 + ev * s
                        acc1 = acc1 + od * s
                g = q2 * 8 + h
                obuf[g, pl.ds(0, 16)] = acc0
                obuf[g, pl.ds(16, 16)] = acc1
            return c2

        lax.fori_loop(0, NH, hloop, 0)

    pltpu.async_copy(idx_src(0), ms[0], smi[0])
    pltpu.async_copy(w_src(0), wv[0], smw[0])
    pltpu.async_copy(idx_src(1), ms[1], smi[1])
    pltpu.async_copy(w_src(1), wv[1], smw[1])
    pltpu.make_async_copy(idx_src(0), ms[0], smi[0]).wait()
    fire(ms[0], rs[0], sg[0])

    def iter_t(t, carry):
        s0 = t * 4
        for k in range(4):
            s = s0 + k
            p = k % 2
            ka, kb = (k + 1) % 4, (k + 2) % 4
            pltpu.make_async_copy(idx_src(s + 1), ms[ka], smi[ka]).wait()
            fire(ms[ka], rs[1 - p], sg[1 - p])
            pltpu.async_copy(idx_src(s + 2), ms[kb], smi[kb])
            pltpu.async_copy(w_src(s + 2), wv[kb], smw[kb])
            pltpu.make_async_copy(table.at[pl.ds(0, RPS)], rs[p], sg[p]).wait()
            pltpu.make_async_copy(w_src(s), wv[k], smw[k]).wait()

            @pl.when(s >= 2)
            def _():
                pltpu.make_async_copy(os[p], out_hbm.at[pl.ds(0, G)],
                                      so[p]).wait()

            compute(wv[k], rs[p], os[p])
            pltpu.async_copy(
                os[p],
                out_hbm.at[pl.ds(pl.multiple_of(obase + s * G, 8), G)], so[p])
        return carry

    lax.fori_loop(0, STEPS // 4, iter_t, 0)
    pltpu.make_async_copy(table.at[pl.ds(0, RPS)], rs[0], sg[0]).wait()
    pltpu.make_async_copy(idx_src(0), ms[1], smi[1]).wait()
    pltpu.make_async_copy(w_src(0), wv[0], smw[0]).wait()
    pltpu.make_async_copy(w_src(0), wv[1], smw[1]).wait()
    pltpu.make_async_copy(os[0], out_hbm.at[pl.ds(0, G)], so[0]).wait()
    pltpu.make_async_copy(os[1], out_hbm.at[pl.ds(0, G)], so[1]).wait()


def kernel(query, reference_points, value, spatial_shapes, level_start_index,
           W_off, b_off, W_attn, b_attn, W_val, b_val, W_out, b_out):
    f32 = jnp.float32

    VT = 640
    valproj = pl.pallas_call(
        _valproj_body,
        grid=(B * LV // VT,),
        in_specs=[
            pl.BlockSpec((VT, C), lambda i: (i, 0)),
            pl.BlockSpec((C, C), lambda i: (0, 0)),
            pl.BlockSpec((1, C), lambda i: (0, 0)),
        ],
        out_specs=pl.BlockSpec((VT, C), lambda i: (i, 0)),
        out_shape=jax.ShapeDtypeStruct((B * LV, C), jnp.bfloat16),
    )(value.reshape(B * LV, C), W_val, b_val.reshape(1, C))
    table = valproj.reshape(B * LV * NH, D)

    woffp = W_off.reshape(C, NH, NL, NP, 2).transpose(0, 4, 1, 2, 3)
    woffp = woffp.reshape(C, C)
    boffp = b_off.reshape(NH, NL, NP, 2).transpose(3, 0, 1, 2).reshape(1, C)
    scale = jnp.asarray(_SS[:, ::-1].astype(np.float32))
    rp8 = (reference_points * scale).transpose(0, 1, 3, 2).reshape(B, LQ, 8)
    nrows = GROUPS * 64 // 128
    idx2d, w2d = pl.pallas_call(
        _sample_body,
        grid=(B, NQT),
        in_specs=[
            pl.BlockSpec((1, QT, C), lambda b, i: (b, i, 0)),
            pl.BlockSpec((1, QT, 8), lambda b, i: (b, i, 0)),
            pl.BlockSpec((C, C), lambda b, i: (0, 0)),
            pl.BlockSpec((1, C), lambda b, i: (0, 0)),
            pl.BlockSpec((C, NH * 16), lambda b, i: (0, 0)),
            pl.BlockSpec((1, NH * 16), lambda b, i: (0, 0)),
        ],
        out_specs=[
            pl.BlockSpec((QT * 4, 128), lambda b, i: (b * NQT + i, 0)),
            pl.BlockSpec((QT * 4, 128), lambda b, i: (b * NQT + i, 0)),
        ],
        out_shape=[
            jax.ShapeDtypeStruct((nrows, 128), jnp.int32),
            jax.ShapeDtypeStruct((nrows, 128), f32),
        ],
    )(query, rp8, woffp, boffp, W_attn, b_attn.reshape(1, NH * 16))

    mesh = plsc.VectorSubcoreMesh(core_axis_name="c", subcore_axis_name="s",
                                  num_cores=NC, num_subcores=NS)
    sc = pl.kernel(
        _sc_body,
        out_type=jax.ShapeDtypeStruct((GROUPS, D), f32),
        mesh=mesh,
        compiler_params=pltpu.CompilerParams(needs_layout_passes=False,
                                             use_tc_tiling_on_sc=False),
        scratch_types=(
            [pltpu.VMEM((IDX_ROWS, 128), jnp.int32)] * 4
            + [pltpu.VMEM((IDX_ROWS, 128), f32)] * 4
            + [pltpu.VMEM((RPS, D), jnp.bfloat16)] * 2
            + [pltpu.VMEM((G, D), f32)] * 2
            + [pltpu.SemaphoreType.DMA] * 12
        ),
    )
    sampled = sc(table, idx2d, w2d)

    perm = np.concatenate([np.arange(0, D, 2), np.arange(1, D, 2)])
    W_outp = W_out.reshape(NH, D, C)[:, perm, :].reshape(C, C)
    out = pl.pallas_call(
        _outproj_body,
        grid=(B * LQ // QT,),
        in_specs=[
            pl.BlockSpec((QT, C), lambda i: (i, 0)),
            pl.BlockSpec((C, C), lambda i: (0, 0)),
            pl.BlockSpec((1, C), lambda i: (0, 0)),
        ],
        out_specs=pl.BlockSpec((QT, C), lambda i: (i, 0)),
        out_shape=jax.ShapeDtypeStruct((B * LQ, C), f32),
    )(sampled.reshape(B * LQ, C), W_outp, b_out.reshape(1, C))
    return out.reshape(B, LQ, C)


# speedup vs baseline: 200.7454x; 1.0622x over previous
import functools

import jax
import jax.numpy as jnp
import numpy as np
from jax import lax
from jax.experimental import pallas as pl
from jax.experimental.pallas import tpu as pltpu
from jax.experimental.pallas import tpu_sc as plsc

_SS = np.array([[64, 64], [32, 32], [16, 16], [8, 8]], dtype=np.int64)
_AREAS = _SS[:, 0] * _SS[:, 1]
_LSI = np.concatenate([np.zeros(1, dtype=np.int64), np.cumsum(_AREAS)[:-1]])
LV = int(_AREAS.sum())
LQ = LV
B = 4
C = 256
NH, NL, NP = 8, 4, 4
D = C // NH

QT = 680
NQT = LQ // QT

NC, NS = 2, 16
NW = NC * NS
GROUPS = B * LQ * NH
GPW = GROUPS // NW
G = 32
STEPS = GPW // G
RPS = G * NL * NP * 4
IDX_ROWS = RPS // 128

def _valproj_body(v_ref, w_ref, b_ref, o_ref):
    o_ref[...] = (jnp.dot(v_ref[...].astype(jnp.bfloat16),
                          w_ref[...].astype(jnp.bfloat16),
                          preferred_element_type=jnp.float32)
                  + b_ref[...]).astype(jnp.bfloat16)


def _outproj_body(x_ref, w_ref, b_ref, o_ref):
    o_ref[...] = jnp.dot(x_ref[...].astype(jnp.bfloat16),
                         w_ref[...].astype(jnp.bfloat16),
                         preferred_element_type=jnp.float32) + b_ref[...]


def _sample_body(q_ref, rp_ref, woff_ref, boff_ref, wattn_ref, battn_ref,
                 idx_ref, w_ref):
    i32, f32 = jnp.int32, jnp.float32
    b = pl.program_id(0)
    q = q_ref[0]
    off = jnp.dot(q, woff_ref[...],
                  preferred_element_type=f32) + boff_ref[...]
    attn = jnp.dot(q, wattn_ref[...],
                   preferred_element_type=f32) + battn_ref[...]
    m = jnp.max(attn, axis=1, keepdims=True)
    e = jnp.exp(attn - m)
    r0 = lax.broadcasted_iota(i32, (128, 128), 0)
    c0 = lax.broadcasted_iota(i32, (128, 128), 1)
    seg = (r0 // 16 == c0 // 16).astype(f32)
    denom = jnp.dot(e, seg, preferred_element_type=f32)
    aw = e / denom
    rp8 = rp_ref[0]
    kio = lax.broadcasted_iota(i32, (1, 128), 1)
    lidx = (kio % 16) // 4
    h_lane = kio // 16
    wrow_i = lax.shift_right_logical(jnp.full((1, 128), 64, i32), lidx)
    wrow = wrow_i.astype(f32)
    base_c = jnp.where(
        lidx == 0, 0,
        jnp.where(lidx == 1, int(_LSI[1]) * NH,
                  jnp.where(lidx == 2, int(_LSI[2]) * NH,
                            int(_LSI[3]) * NH))) + h_lane
    xb = jnp.zeros((QT, 128), f32)
    yb = jnp.zeros((QT, 128), f32)
    for l in range(NL):
        msk = (lidx == l).astype(f32)
        xb = xb + msk * rp8[:, l:l + 1]
        yb = yb + msk * rp8[:, NL + l:NL + l + 1]
    x = xb + off[:, :128] - 0.5
    y = yb + off[:, 128:] - 0.5
    x0 = jnp.floor(x)
    y0 = jnp.floor(y)
    fx = x - x0
    fy = y - y0
    vx0 = (x0 >= 0.0) & (x0 <= wrow - 1.0)
    vx1 = (x0 + 1.0 >= 0.0) & (x0 + 1.0 <= wrow - 1.0)
    vy0 = (y0 >= 0.0) & (y0 <= wrow - 1.0)
    vy1 = (y0 + 1.0 >= 0.0) & (y0 + 1.0 <= wrow - 1.0)
    xc0 = jnp.clip(x0, 0.0, wrow - 1.0).astype(i32)
    xc1 = jnp.clip(x0 + 1.0, 0.0, wrow - 1.0).astype(i32)
    yc0 = jnp.clip(y0, 0.0, wrow - 1.0).astype(i32)
    yc1 = jnp.clip(y0 + 1.0, 0.0, wrow - 1.0).astype(i32)
    base = base_c + b * (LQ * NH)
    i00 = base + (yc0 * wrow_i + xc0) * NH
    dx = (xc1 - xc0) * NH
    dy = (yc1 - yc0) * (wrow_i * NH)
    i10 = i00 + dx
    i01 = i00 + dy
    i11 = i01 + dx
    ax0 = (1.0 - fx) * vx0.astype(f32) * aw
    ax1 = fx * vx1.astype(f32) * aw
    by0 = (1.0 - fy) * vy0.astype(f32)
    by1 = fy * vy1.astype(f32)
    w00 = ax0 * by0
    w10 = ax1 * by0
    w01 = ax0 * by1
    w11 = ax1 * by1
    icat = jnp.concatenate([t.reshape(QT, 1, 128) for t in
                            (i00, i10, i01, i11)], axis=1)
    wcat = jnp.concatenate([t.reshape(QT, 1, 128) for t in
                            (w00, w10, w01, w11)], axis=1)
    idx_ref[...] = icat.reshape(QT * 4, 128)
    w_ref[...] = wcat.reshape(QT * 4, 128)


def _sc_body(table, idx_hbm, w_hbm, out_hbm,
             m0, m1, w0, w1, r0, r1, o0, o1,
             smi0, smi1, smw0, smw1, sg0, sg1, so0, so1):
    wid = lax.axis_index("s") * NC + lax.axis_index("c")
    ms = [m0, m1]
    wv = [w0, w1]
    rs = [r0, r1]
    os = [o0, o1]
    smi = [smi0, smi1]
    smw = [smw0, smw1]
    sg = [sg0, sg1]
    so = [so0, so1]
    ibase = wid * (GPW * 64 // 128)
    obase = wid * GPW

    def idx_src(s):
        s = jnp.minimum(s, STEPS - 1)
        return idx_hbm.at[pl.ds(pl.multiple_of(ibase + s * IDX_ROWS, 8),
                                IDX_ROWS)]

    def w_src(s):
        s = jnp.minimum(s, STEPS - 1)
        return w_hbm.at[pl.ds(pl.multiple_of(ibase + s * IDX_ROWS, 8),
                              IDX_ROWS)]

    def fire(mbuf, rbuf, sem):
        for j in range(IDX_ROWS):
            pltpu.async_copy(table.at[mbuf.at[j]],
                             rbuf.at[pl.ds(j * 128, 128)], sem)

    gdn = lax.GatherDimensionNumbers(offset_dims=(), collapsed_slice_dims=(0,),
                                     start_index_map=(0,))

    def splat(v, j):
        idx = jnp.full((16, 1), j, jnp.int32)
        return lax.gather(v, idx, gdn, (1,),
                          mode=lax.GatherScatterMode.PROMISE_IN_BOUNDS)

    def compute(wbuf, rbuf, obuf):
        def hloop(h, c2):
            hb = h * 16
            for q4 in range(4):
                acc0 = jnp.zeros((16,), jnp.float32)
                acc1 = jnp.zeros((16,), jnp.float32)
                for c in range(NL):
                    w16 = wbuf[q4 * 4 + c, pl.ds(hb, 16)]
                    for lp in range(16):
                        sv = splat(w16, lp)
                        r = q4 * 512 + c * 128 + hb + lp
                        ev, od = plsc.unpack(rbuf[r],
                                             format=plsc.PackFormat.INTERLEAVED)
                        acc0 = acc0 + ev * sv
                        acc1 = acc1 + od * sv
                g = q4 * 8 + h
                obuf[g, pl.ds(0, 16)] = acc0
                obuf[g, pl.ds(16, 16)] = acc1
            return c2

        lax.fori_loop(0, NH, hloop, 0)

    pltpu.async_copy(idx_src(0), ms[0], smi[0])
    pltpu.async_copy(w_src(0), wv[0], smw[0])
    pltpu.async_copy(idx_src(1), ms[1], smi[1])
    pltpu.async_copy(w_src(1), wv[1], smw[1])
    pltpu.make_async_copy(idx_src(0), ms[0], smi[0]).wait()
    fire(ms[0], rs[0], sg[0])

    def iter_t(t, carry):
        s0 = t * 2
        for k in range(2):
            s = s0 + k
            kb = 1 - k
            pltpu.make_async_copy(idx_src(s + 1), ms[kb], smi[kb]).wait()
            fire(ms[kb], rs[kb], sg[kb])
            pltpu.make_async_copy(table.at[pl.ds(0, RPS)], rs[k], sg[k]).wait()
            pltpu.make_async_copy(w_src(s), wv[k], smw[k]).wait()
            pltpu.async_copy(idx_src(s + 2), ms[k], smi[k])

            @pl.when(s >= 2)
            def _():
                pltpu.make_async_copy(os[k], out_hbm.at[pl.ds(0, G)],
                                      so[k]).wait()

            compute(wv[k], rs[k], os[k])
            pltpu.async_copy(w_src(s + 2), wv[k], smw[k])
            pltpu.async_copy(
                os[k],
                out_hbm.at[pl.ds(pl.multiple_of(obase + s * G, 8), G)], so[k])
        return carry

    lax.fori_loop(0, STEPS // 2, iter_t, 0)
    pltpu.make_async_copy(table.at[pl.ds(0, RPS)], rs[0], sg[0]).wait()
    pltpu.make_async_copy(idx_src(0), ms[1], smi[1]).wait()
    pltpu.make_async_copy(w_src(0), wv[0], smw[0]).wait()
    pltpu.make_async_copy(w_src(0), wv[1], smw[1]).wait()
    pltpu.make_async_copy(os[0], out_hbm.at[pl.ds(0, G)], so[0]).wait()
    pltpu.make_async_copy(os[1], out_hbm.at[pl.ds(0, G)], so[1]).wait()


def kernel(query, reference_points, value, spatial_shapes, level_start_index,
           W_off, b_off, W_attn, b_attn, W_val, b_val, W_out, b_out):
    f32 = jnp.float32

    VT = 640
    valproj = pl.pallas_call(
        _valproj_body,
        grid=(B * LV // VT,),
        in_specs=[
            pl.BlockSpec((VT, C), lambda i: (i, 0)),
            pl.BlockSpec((C, C), lambda i: (0, 0)),
            pl.BlockSpec((1, C), lambda i: (0, 0)),
        ],
        out_specs=pl.BlockSpec((VT, C), lambda i: (i, 0)),
        out_shape=jax.ShapeDtypeStruct((B * LV, C), jnp.bfloat16),
    )(value.reshape(B * LV, C), W_val, b_val.reshape(1, C))
    table = valproj.reshape(B * LV * NH, D)

    woffp = W_off.reshape(C, NH, NL, NP, 2).transpose(0, 4, 1, 2, 3)
    woffp = woffp.reshape(C, C)
    boffp = b_off.reshape(NH, NL, NP, 2).transpose(3, 0, 1, 2).reshape(1, C)
    scale = jnp.asarray(_SS[:, ::-1].astype(np.float32))
    rp8 = (reference_points * scale).transpose(0, 1, 3, 2).reshape(B, LQ, 8)
    nrows = GROUPS * 64 // 128
    idx2d, w2d = pl.pallas_call(
        _sample_body,
        grid=(B, NQT),
        in_specs=[
            pl.BlockSpec((1, QT, C), lambda b, i: (b, i, 0)),
            pl.BlockSpec((1, QT, 8), lambda b, i: (b, i, 0)),
            pl.BlockSpec((C, C), lambda b, i: (0, 0)),
            pl.BlockSpec((1, C), lambda b, i: (0, 0)),
            pl.BlockSpec((C, NH * 16), lambda b, i: (0, 0)),
            pl.BlockSpec((1, NH * 16), lambda b, i: (0, 0)),
        ],
        out_specs=[
            pl.BlockSpec((QT * 4, 128), lambda b, i: (b * NQT + i, 0)),
            pl.BlockSpec((QT * 4, 128), lambda b, i: (b * NQT + i, 0)),
        ],
        out_shape=[
            jax.ShapeDtypeStruct((nrows, 128), jnp.int32),
            jax.ShapeDtypeStruct((nrows, 128), f32),
        ],
    )(query, rp8, woffp, boffp, W_attn, b_attn.reshape(1, NH * 16))

    mesh = plsc.VectorSubcoreMesh(core_axis_name="c", subcore_axis_name="s",
                                  num_cores=NC, num_subcores=NS)
    sc = pl.kernel(
        _sc_body,
        out_type=jax.ShapeDtypeStruct((GROUPS, D), f32),
        mesh=mesh,
        compiler_params=pltpu.CompilerParams(needs_layout_passes=False,
                                             use_tc_tiling_on_sc=False),
        scratch_types=(
            [pltpu.VMEM((IDX_ROWS, 128), jnp.int32)] * 2
            + [pltpu.VMEM((IDX_ROWS, 128), f32)] * 2
            + [pltpu.VMEM((RPS, D), jnp.bfloat16)] * 2
            + [pltpu.VMEM((G, D), f32)] * 2
            + [pltpu.SemaphoreType.DMA] * 8
        ),
    )
    sampled = sc(table, idx2d, w2d)

    perm = np.concatenate([np.arange(0, D, 2), np.arange(1, D, 2)])
    W_outp = W_out.reshape(NH, D, C)[:, perm, :].reshape(C, C)
    out = pl.pallas_call(
        _outproj_body,
        grid=(B * LQ // QT,),
        in_specs=[
            pl.BlockSpec((QT, C), lambda i: (i, 0)),
            pl.BlockSpec((C, C), lambda i: (0, 0)),
            pl.BlockSpec((1, C), lambda i: (0, 0)),
        ],
        out_specs=pl.BlockSpec((QT, C), lambda i: (i, 0)),
        out_shape=jax.ShapeDtypeStruct((B * LQ, C), f32),
    )(sampled.reshape(B * LQ, C), W_outp, b_out.reshape(1, C))
    return out.reshape(B, LQ, C)


# speedup vs baseline: 201.1005x; 1.0018x over previous
import jax
import jax.numpy as jnp
import numpy as np
from jax import lax
from jax.experimental import pallas as pl
from jax.experimental.pallas import tpu as pltpu
from jax.experimental.pallas import tpu_sc as plsc

_SS = np.array([[64, 64], [32, 32], [16, 16], [8, 8]], dtype=np.int64)
_AREAS = _SS[:, 0] * _SS[:, 1]
_LSI = np.concatenate([np.zeros(1, dtype=np.int64), np.cumsum(_AREAS)[:-1]])
LV = int(_AREAS.sum())
LQ = LV
B = 4
C = 256
NH, NL, NP = 8, 4, 4
D = C // NH

QT = 680
NQT = LQ // QT

NC, NS = 2, 16
NW = NC * NS
GROUPS = B * LQ * NH
GPW = GROUPS // NW
G = 32
STEPS = GPW // G
RPS = G * NL * NP * 4
IDX_ROWS = RPS // 128

def _valproj_body(v_ref, w_ref, b_ref, o_ref):
    o_ref[...] = (jnp.dot(v_ref[...].astype(jnp.bfloat16),
                          w_ref[...].astype(jnp.bfloat16),
                          preferred_element_type=jnp.float32)
                  + b_ref[...]).astype(jnp.bfloat16)


def _outproj_body(x_ref, w_ref, b_ref, o_ref):
    o_ref[...] = jnp.dot(x_ref[...].astype(jnp.bfloat16),
                         w_ref[...].astype(jnp.bfloat16),
                         preferred_element_type=jnp.float32) + b_ref[...]


def _sample_body(q_ref, rp_ref, woff_ref, boff_ref, wattn_ref, battn_ref,
                 idx_ref, w_ref):
    i32, f32 = jnp.int32, jnp.float32
    b = pl.program_id(0)
    q = q_ref[0]
    off = jnp.dot(q, woff_ref[...],
                  preferred_element_type=f32) + boff_ref[...]
    attn = jnp.dot(q, wattn_ref[...],
                   preferred_element_type=f32) + battn_ref[...]
    m = jnp.max(attn, axis=1, keepdims=True)
    e = jnp.exp(attn - m)
    r0 = lax.broadcasted_iota(i32, (128, 128), 0)
    c0 = lax.broadcasted_iota(i32, (128, 128), 1)
    seg = (r0 // 16 == c0 // 16).astype(f32)
    denom = jnp.dot(e, seg, preferred_element_type=f32)
    aw = e / denom
    rp8 = rp_ref[0]
    kio = lax.broadcasted_iota(i32, (1, 128), 1)
    lidx = (kio % 16) // 4
    h_lane = kio // 16
    wrow_i = lax.shift_right_logical(jnp.full((1, 128), 64, i32), lidx)
    wrow = wrow_i.astype(f32)
    base_c = jnp.where(
        lidx == 0, 0,
        jnp.where(lidx == 1, int(_LSI[1]) * NH,
                  jnp.where(lidx == 2, int(_LSI[2]) * NH,
                            int(_LSI[3]) * NH))) + h_lane
    xb = jnp.zeros((QT, 128), f32)
    yb = jnp.zeros((QT, 128), f32)
    for l in range(NL):
        msk = (lidx == l).astype(f32)
        xb = xb + msk * rp8[:, l:l + 1]
        yb = yb + msk * rp8[:, NL + l:NL + l + 1]
    x = xb + off[:, :128] - 0.5
    y = yb + off[:, 128:] - 0.5
    x0 = jnp.floor(x)
    y0 = jnp.floor(y)
    fx = x - x0
    fy = y - y0
    vx0 = (x0 >= 0.0) & (x0 <= wrow - 1.0)
    vx1 = (x0 + 1.0 >= 0.0) & (x0 + 1.0 <= wrow - 1.0)
    vy0 = (y0 >= 0.0) & (y0 <= wrow - 1.0)
    vy1 = (y0 + 1.0 >= 0.0) & (y0 + 1.0 <= wrow - 1.0)
    xc0 = jnp.clip(x0, 0.0, wrow - 1.0).astype(i32)
    xc1 = jnp.clip(x0 + 1.0, 0.0, wrow - 1.0).astype(i32)
    yc0 = jnp.clip(y0, 0.0, wrow - 1.0).astype(i32)
    yc1 = jnp.clip(y0 + 1.0, 0.0, wrow - 1.0).astype(i32)
    base = base_c + b * (LQ * NH)
    i00 = base + (yc0 * wrow_i + xc0) * NH
    dx = (xc1 - xc0) * NH
    dy = (yc1 - yc0) * (wrow_i * NH)
    i10 = i00 + dx
    i01 = i00 + dy
    i11 = i01 + dx
    ax0 = (1.0 - fx) * vx0.astype(f32) * aw
    ax1 = fx * vx1.astype(f32) * aw
    by0 = (1.0 - fy) * vy0.astype(f32)
    by1 = fy * vy1.astype(f32)
    w00 = ax0 * by0
    w10 = ax1 * by0
    w01 = ax0 * by1
    w11 = ax1 * by1
    icat = jnp.concatenate([t.reshape(QT, 1, 128) for t in
                            (i00, i10, i01, i11)], axis=1)
    wcat = jnp.concatenate([t.reshape(QT, 1, 128) for t in
                            (w00, w10, w01, w11)], axis=1)
    idx_ref[...] = icat.reshape(QT * 4, 128)
    w_ref[...] = wcat.reshape(QT * 4, 128)


def _sc_body(table, idx_hbm, w_hbm, out_hbm,
             m0, m1, w0, w1, r0, r1, o0, o1,
             smi0, smi1, smw0, smw1, sg0, sg1, so0, so1):
    wid = lax.axis_index("s") * NC + lax.axis_index("c")
    ms = [m0, m1]
    wv = [w0, w1]
    rs = [r0, r1]
    os = [o0, o1]
    smi = [smi0, smi1]
    smw = [smw0, smw1]
    sg = [sg0, sg1]
    so = [so0, so1]
    ibase = wid * (GPW * 64 // 128)
    obase = wid * GPW

    def idx_src(s):
        s = jnp.minimum(s, STEPS - 1)
        return idx_hbm.at[pl.ds(pl.multiple_of(ibase + s * IDX_ROWS, 8),
                                IDX_ROWS)]

    def w_src(s):
        s = jnp.minimum(s, STEPS - 1)
        return w_hbm.at[pl.ds(pl.multiple_of(ibase + s * IDX_ROWS, 8),
                              IDX_ROWS)]

    def fire(mbuf, rbuf, sem):
        for j in range(IDX_ROWS):
            pltpu.async_copy(table.at[mbuf.at[j]],
                             rbuf.at[pl.ds(j * 128, 128)], sem)

    gdn = lax.GatherDimensionNumbers(offset_dims=(), collapsed_slice_dims=(0,),
                                     start_index_map=(0,))

    def splat(v, j):
        idx = jnp.full((16, 1), j, jnp.int32)
        return lax.gather(v, idx, gdn, (1,),
                          mode=lax.GatherScatterMode.PROMISE_IN_BOUNDS)

    def compute(wbuf, rbuf, obuf):
        def hloop(h, c2):
            hb = h * 16
            for q4 in range(4):
                acc0 = jnp.zeros((16,), jnp.float32)
                acc1 = jnp.zeros((16,), jnp.float32)
                for c in range(NL):
                    w16 = wbuf[q4 * 4 + c, pl.ds(hb, 16)]
                    for lp in range(16):
                        sv = splat(w16, lp)
                        r = q4 * 512 + c * 128 + hb + lp
                        ev, od = plsc.unpack(rbuf[r],
                                             format=plsc.PackFormat.INTERLEAVED)
                        acc0 = acc0 + ev * sv
                        acc1 = acc1 + od * sv
                g = q4 * 8 + h
                obuf[g, pl.ds(0, 16)] = acc0
                obuf[g, pl.ds(16, 16)] = acc1
            return c2

        lax.fori_loop(0, NH, hloop, 0)

    pltpu.async_copy(idx_src(0), ms[0], smi[0])
    pltpu.async_copy(w_src(0), wv[0], smw[0])
    pltpu.async_copy(idx_src(1), ms[1], smi[1])
    pltpu.async_copy(w_src(1), wv[1], smw[1])
    pltpu.make_async_copy(idx_src(0), ms[0], smi[0]).wait()
    fire(ms[0], rs[0], sg[0])

    def iter_t(t, carry):
        s0 = t * 2
        for k in range(2):
            s = s0 + k
            kb = 1 - k
            pltpu.make_async_copy(idx_src(s + 1), ms[kb], smi[kb]).wait()
            fire(ms[kb], rs[kb], sg[kb])
            pltpu.make_async_copy(table.at[pl.ds(0, RPS)], rs[k], sg[k]).wait()
            pltpu.make_async_copy(w_src(s), wv[k], smw[k]).wait()
            pltpu.async_copy(idx_src(s + 2), ms[k], smi[k])

            @pl.when(s >= 2)
            def _():
                pltpu.make_async_copy(os[k], out_hbm.at[pl.ds(0, G)],
                                      so[k]).wait()

            compute(wv[k], rs[k], os[k])
            pltpu.async_copy(w_src(s + 2), wv[k], smw[k])
            pltpu.async_copy(
                os[k],
                out_hbm.at[pl.ds(pl.multiple_of(obase + s * G, 8), G)], so[k])
        return carry

    lax.fori_loop(0, STEPS // 2, iter_t, 0)
    pltpu.make_async_copy(table.at[pl.ds(0, RPS)], rs[0], sg[0]).wait()
    pltpu.make_async_copy(idx_src(0), ms[1], smi[1]).wait()
    pltpu.make_async_copy(w_src(0), wv[0], smw[0]).wait()
    pltpu.make_async_copy(w_src(0), wv[1], smw[1]).wait()
    pltpu.make_async_copy(os[0], out_hbm.at[pl.ds(0, G)], so[0]).wait()
    pltpu.make_async_copy(os[1], out_hbm.at[pl.ds(0, G)], so[1]).wait()


def kernel(query, reference_points, value, spatial_shapes, level_start_index,
           W_off, b_off, W_attn, b_attn, W_val, b_val, W_out, b_out):
    f32 = jnp.float32

    VT = 640
    valproj = pl.pallas_call(
        _valproj_body,
        grid=(B * LV // VT,),
        in_specs=[
            pl.BlockSpec((VT, C), lambda i: (i, 0)),
            pl.BlockSpec((C, C), lambda i: (0, 0)),
            pl.BlockSpec((1, C), lambda i: (0, 0)),
        ],
        out_specs=pl.BlockSpec((VT, C), lambda i: (i, 0)),
        out_shape=jax.ShapeDtypeStruct((B * LV, C), jnp.bfloat16),
    )(value.reshape(B * LV, C), W_val, b_val.reshape(1, C))
    table = valproj.reshape(B * LV * NH, D)

    woffp = W_off.reshape(C, NH, NL, NP, 2).transpose(0, 4, 1, 2, 3)
    woffp = woffp.reshape(C, C)
    boffp = b_off.reshape(NH, NL, NP, 2).transpose(3, 0, 1, 2).reshape(1, C)
    scale = jnp.asarray(_SS[:, ::-1].astype(np.float32))
    rp8 = (reference_points * scale).transpose(0, 1, 3, 2).reshape(B, LQ, 8)
    nrows = GROUPS * 64 // 128
    idx2d, w2d = pl.pallas_call(
        _sample_body,
        grid=(B, NQT),
        in_specs=[
            pl.BlockSpec((1, QT, C), lambda b, i: (b, i, 0)),
            pl.BlockSpec((1, QT, 8), lambda b, i: (b, i, 0)),
            pl.BlockSpec((C, C), lambda b, i: (0, 0)),
            pl.BlockSpec((1, C), lambda b, i: (0, 0)),
            pl.BlockSpec((C, NH * 16), lambda b, i: (0, 0)),
            pl.BlockSpec((1, NH * 16), lambda b, i: (0, 0)),
        ],
        out_specs=[
            pl.BlockSpec((QT * 4, 128), lambda b, i: (b * NQT + i, 0)),
            pl.BlockSpec((QT * 4, 128), lambda b, i: (b * NQT + i, 0)),
        ],
        out_shape=[
            jax.ShapeDtypeStruct((nrows, 128), jnp.int32),
            jax.ShapeDtypeStruct((nrows, 128), f32),
        ],
    )(query, rp8, woffp, boffp, W_attn, b_attn.reshape(1, NH * 16))

    mesh = plsc.VectorSubcoreMesh(core_axis_name="c", subcore_axis_name="s",
                                  num_cores=NC, num_subcores=NS)
    sc = pl.kernel(
        _sc_body,
        out_type=jax.ShapeDtypeStruct((GROUPS, D), f32),
        mesh=mesh,
        compiler_params=pltpu.CompilerParams(needs_layout_passes=False,
                                             use_tc_tiling_on_sc=False),
        scratch_types=(
            [pltpu.VMEM((IDX_ROWS, 128), jnp.int32)] * 2
            + [pltpu.VMEM((IDX_ROWS, 128), f32)] * 2
            + [pltpu.VMEM((RPS, D), jnp.bfloat16)] * 2
            + [pltpu.VMEM((G, D), f32)] * 2
            + [pltpu.SemaphoreType.DMA] * 8
        ),
    )
    sampled = sc(table, idx2d, w2d)

    perm = np.concatenate([np.arange(0, D, 2), np.arange(1, D, 2)])
    W_outp = W_out.reshape(NH, D, C)[:, perm, :].reshape(C, C)
    out = pl.pallas_call(
        _outproj_body,
        grid=(B * LQ // QT,),
        in_specs=[
            pl.BlockSpec((QT, C), lambda i: (i, 0)),
            pl.BlockSpec((C, C), lambda i: (0, 0)),
            pl.BlockSpec((1, C), lambda i: (0, 0)),
        ],
        out_specs=pl.BlockSpec((QT, C), lambda i: (i, 0)),
        out_shape=jax.ShapeDtypeStruct((B * LQ, C), f32),
    )(sampled.reshape(B * LQ, C), W_outp, b_out.reshape(1, C))
    return out.reshape(B, LQ, C)
